# baseline (speedup 1.0000x reference)
import functools

import jax
import jax.numpy as jnp
from jax import lax
from jax.experimental import pallas as pl
from jax.experimental.pallas import tpu as pltpu
from jax.experimental.pallas import tpu_sc as plsc

N = 10000
E = 160000
D = 256
HALF = 128
H = 512
NRBF = 16
NSPEC = 100
DOFF = 64
DON = 64
CUTOFF = 5.0
F32 = jnp.float32
I32 = jnp.int32

NC, NS = 2, 16
N_PAD = 10240
NODES_PER_TILE = N_PAD // NS
E_PER_TILE_A = E // NS
CH_A = 80
IT_A = E_PER_TILE_A // CH_A
CH_C = 64
NCHUNK_C = E // CH_C
IT_C = -(-NCHUNK_C // (NC * NS))
EB = 512
EBF = 640
NSPLIT = 1


def _axis(name):
    return lax.axis_index(name)


def _gather_rows(tbl_h, idx_ref, dst, sem):
    pltpu.async_copy(tbl_h.at[idx_ref], dst, sem).wait()


def _gather_start(tbl_h, idx_ref, dst, sem):
    return pltpu.async_copy(tbl_h.at[idx_ref], dst, sem)


def _scatter_add_rows(src, acc, idx_ref):
    pltpu.sync_copy(src, acc.at[idx_ref], add=True)


def _cutoff_fn(r):
    return jnp.where(r < CUTOFF, 0.5 * (jnp.cos(jnp.pi * r / CUTOFF) + 1.0), 0.0)


def _radial(disp, wrbf_ref):
    r = jnp.sqrt(jnp.sum(disp * disp, axis=1, keepdims=True))
    mu = lax.broadcasted_iota(I32, (1, NRBF), 1).astype(F32) * (CUTOFF / (NRBF - 1))
    phi = jnp.exp(-10.0 * (r - mu) ** 2)
    cut = _cutoff_fn(r)
    w = jnp.dot(phi, wrbf_ref[...], preferred_element_type=F32)
    return w, cut


EPW = E // (NC * NS)
EPW_PAD = EPW + 8


def _zjk_body(an_h, acj_h, zj_h, an_v, jv, zb):
    c = _axis("c")
    s = _axis("s")
    w = s * NC + c
    pltpu.sync_copy(an_h, an_v)
    base = w * EPW
    pltpu.sync_copy(acj_h.at[pl.ds(base, EPW)], jv.at[pl.ds(0, EPW)])
    lane = lax.iota(I32, 16)

    @pl.loop(0, EPW_PAD // 16)
    def _(k):
        off = k * 16
        jj = jv[pl.ds(off, 16)]
        jj = jnp.where(lane < EPW - off, jj, 0)
        zb[pl.ds(off, 16)] = plsc.load_gather(an_v, [jj])

    pltpu.sync_copy(zb.at[pl.ds(0, EPW)], zj_h.at[pl.ds(base, EPW)])


def _zjk(an_pad, acj):
    mesh = plsc.VectorSubcoreMesh(
        core_axis_name="c", subcore_axis_name="s", num_cores=NC, num_subcores=NS)
    f = pl.kernel(
        _zjk_body,
        out_type=jax.ShapeDtypeStruct((E,), I32),
        mesh=mesh,
        compiler_params=pltpu.CompilerParams(needs_layout_passes=False),
        scratch_types=[
            pltpu.VMEM((N_PAD,), I32),
            pltpu.VMEM((EPW_PAD,), I32),
            pltpu.VMEM((EPW_PAD,), I32),
        ],
    )
    return f(an_pad, acj)


WB = 640


def _msg_body(disp_ref, zj_ref, wrbf_ref, zpad_ref, out_l_ref, out_r_ref):
    w, cut = _radial(disp_ref[...], wrbf_ref)
    z = zj_ref[...]
    oh = (z == lax.broadcasted_iota(I32, (1, 128), 1)).astype(F32)
    zq = zpad_ref[...]
    hi = lax.bitcast_convert_type(
        lax.bitcast_convert_type(zq, I32) & jnp.int32(-65536), F32)
    lo = zq - hi
    wc = w * cut
    m = (wc * jnp.dot(oh, hi, preferred_element_type=F32)
         + wc * jnp.dot(oh, lo, preferred_element_type=F32))
    out_l_ref[...] = m[:, :HALF]
    out_r_ref[...] = m[:, HALF:]


def _msg(disp, zj2, wrbf, zpad):
    return pl.pallas_call(
        _msg_body,
        grid=(E // WB,),
        in_specs=[
            pl.BlockSpec((WB, 3), lambda i: (i, 0)),
            pl.BlockSpec((WB, 1), lambda i: (i, 0)),
            pl.BlockSpec((NRBF, D), lambda i: (0, 0)),
            pl.BlockSpec((128, D), lambda i: (0, 0)),
        ],
        out_specs=[pl.BlockSpec((WB, HALF), lambda i: (i, 0))] * 2,
        out_shape=[jax.ShapeDtypeStruct((E, HALF), F32)] * 2,
    )(disp, zj2, wrbf, zpad)


def _acd_body(an_h, aci_h, zl_h, zr_h, ml_h, mr_h, sont_h, shont_h,
              acdl_h, acdr_h, son_h, shon_h,
              an_v, sont_v, shont_v, iv0, iv1, iv2, m0, m1, m2,
              sbuf, shbuf, acc, sem0, sem1, sem2):
    c = _axis("c")
    s = _axis("s")
    node_base = s * NODES_PER_TILE

    def half(z_h, m_h, acd_h):
        for k in range(NODES_PER_TILE // CH_A):
            off = node_base + k * CH_A
            pltpu.sync_copy(an_h.at[pl.ds(off, CH_A)], iv0)
            _gather_rows(z_h, iv0, m0, sem0)
            pltpu.sync_copy(m0, acc.at[pl.ds(off, CH_A)])
        plsc.subcore_barrier()
        ebase = s * E_PER_TILE_A

        def start(t, ivb, mb, sm):
            base = ebase + t * CH_A
            pltpu.async_copy(aci_h.at[pl.ds(base, CH_A)], ivb, sm)
            pltpu.async_copy(m_h.at[pl.ds(base, CH_A)], mb, sm)

        def drain(ivb, mb, sm):
            pltpu.make_async_copy(aci_h.at[pl.ds(0, CH_A)], ivb, sm).wait()
            pltpu.make_async_copy(m_h.at[pl.ds(0, CH_A)], mb, sm).wait()

        start(0, iv0, m0, sem0)
        start(1, iv1, m1, sem1)

        @pl.loop(0, IT_A, step=3)
        def _edge_chunk(t):
            drain(iv0, m0, sem0)
            _scatter_add_rows(m0, acc, iv0)

            @pl.when(t + 2 < IT_A)
            def _():
                start(t + 2, iv2, m2, sem2)

            @pl.when(t + 1 < IT_A)
            def _():
                drain(iv1, m1, sem1)
                _scatter_add_rows(m1, acc, iv1)

                @pl.when(t + 3 < IT_A)
                def _():
                    start(t + 3, iv0, m0, sem0)

            @pl.when(t + 2 < IT_A)
            def _():
                drain(iv2, m2, sem2)
                _scatter_add_rows(m2, acc, iv2)

                @pl.when(t + 4 < IT_A)
                def _():
                    start(t + 4, iv1, m1, sem1)

        plsc.subcore_barrier()
        pltpu.sync_copy(acc.at[pl.ds(node_base, NODES_PER_TILE)],
                        acd_h.at[pl.ds(node_base, NODES_PER_TILE)])

    @pl.when(c == 0)
    def _():
        half(zl_h, ml_h, acdl_h)
        pltpu.sync_copy(an_h, an_v)
        pltpu.sync_copy(sont_h, sont_v)
        pltpu.sync_copy(shont_h, shont_v)

        @pl.loop(0, NODES_PER_TILE // 16)
        def _n16(k):
            zn = an_v[pl.ds(node_base + k * 16, 16)]
            sbuf[pl.ds(k * 16, 16)] = plsc.load_gather(sont_v, [zn])
            shbuf[pl.ds(k * 16, 16)] = plsc.load_gather(shont_v, [zn])

        pltpu.sync_copy(sbuf, son_h.at[pl.ds(node_base, NODES_PER_TILE)])
        pltpu.sync_copy(shbuf, shon_h.at[pl.ds(node_base, NODES_PER_TILE)])

    @pl.when(c == 1)
    def _():
        half(zr_h, mr_h, acdr_h)


def _acd(an_pad, aci, zl, zr, ml, mr, sont, shont):
    mesh = plsc.VectorSubcoreMesh(
        core_axis_name="c", subcore_axis_name="s", num_cores=NC, num_subcores=NS)
    f = pl.kernel(
        _acd_body,
        out_type=(
            jax.ShapeDtypeStruct((N_PAD, HALF), F32),
            jax.ShapeDtypeStruct((N_PAD, HALF), F32),
            jax.ShapeDtypeStruct((N_PAD,), F32),
            jax.ShapeDtypeStruct((N_PAD,), F32),
        ),
        mesh=mesh,
        compiler_params=pltpu.CompilerParams(needs_layout_passes=False),
        scratch_types=[
            pltpu.VMEM((N_PAD,), I32),
            pltpu.VMEM((128,), F32),
            pltpu.VMEM((128,), F32),
            pltpu.VMEM((CH_A,), I32),
            pltpu.VMEM((CH_A,), I32),
            pltpu.VMEM((CH_A,), I32),
            pltpu.VMEM((CH_A, HALF), F32),
            pltpu.VMEM((CH_A, HALF), F32),
            pltpu.VMEM((CH_A, HALF), F32),
            pltpu.VMEM((NODES_PER_TILE,), F32),
            pltpu.VMEM((NODES_PER_TILE,), F32),
            pltpu.VMEM_SHARED((N_PAD, HALF), F32),
            pltpu.SemaphoreType.DMA,
            pltpu.SemaphoreType.DMA,
            pltpu.SemaphoreType.DMA,
        ],
    )
    return f(an_pad, aci, zl, zr, ml, mr, sont, shont)


def _bc_body(nchunk, itc, an_h, bci_h, bcj_h, acdl_h, acdr_h, sp_h, shp_h,
             bl_h, br_h, scl_h, shf_h,
             an_v, sp_v, shp_v, ivA, jvA, ivB, jvB,
             gA0l, gA0r, gA1l, gA1r, gB0l, gB0r, gB1l, gB1r,
             scb, shb, semIA, semIB, semGA, semGB):
    c = _axis("c")
    s = _axis("s")
    w = s * NC + c
    pltpu.sync_copy(an_h, an_v)
    pltpu.sync_copy(sp_h, sp_v)
    pltpu.sync_copy(shp_h, shp_v)
    STRIDE = NC * NS

    def start_idx(t, ivb, jvb, sm):
        base = (w + t * STRIDE) * CH_C
        pltpu.async_copy(bci_h.at[pl.ds(base, CH_C)], ivb, sm)
        pltpu.async_copy(bcj_h.at[pl.ds(base, CH_C)], jvb, sm)

    def drain_idx(ivb, jvb, sm):
        pltpu.make_async_copy(bci_h.at[pl.ds(0, CH_C)], ivb, sm).wait()
        pltpu.make_async_copy(bcj_h.at[pl.ds(0, CH_C)], jvb, sm).wait()

    def fire_g(ivb, jvb, g0lb, g0rb, g1lb, g1rb, sm):
        _gather_start(acdl_h, ivb, g0lb, sm)
        _gather_start(acdr_h, ivb, g0rb, sm)
        _gather_start(acdl_h, jvb, g1lb, sm)
        _gather_start(acdr_h, jvb, g1rb, sm)

    def drain_g(ivb, g0lb, g0rb, g1lb, g1rb, sm):
        pltpu.make_async_copy(acdl_h.at[ivb], g0lb, sm).wait()
        pltpu.make_async_copy(acdr_h.at[ivb], g0rb, sm).wait()
        pltpu.make_async_copy(acdl_h.at[ivb], g1lb, sm).wait()
        pltpu.make_async_copy(acdr_h.at[ivb], g1rb, sm).wait()

    def compute(t, ivb, jvb, g0lb, g0rb, g1lb, g1rb):
        base = (w + t * STRIDE) * CH_C

        @pl.loop(0, CH_C)
        def _row(rr):
            for c0 in range(0, HALF, 16):
                g0lb[rr, pl.ds(c0, 16)] = (
                    g0lb[rr, pl.ds(c0, 16)] * g1lb[rr, pl.ds(c0, 16)])
                g0rb[rr, pl.ds(c0, 16)] = (
                    g0rb[rr, pl.ds(c0, 16)] * g1rb[rr, pl.ds(c0, 16)])

        pltpu.sync_copy(g0lb, bl_h.at[pl.ds(base, CH_C)])
        pltpu.sync_copy(g0rb, br_h.at[pl.ds(base, CH_C)])
        for k in range(CH_C // 16):
            ii = ivb[pl.ds(k * 16, 16)]
            jj = jvb[pl.ds(k * 16, 16)]
            zi = plsc.load_gather(an_v, [ii])
            zj = plsc.load_gather(an_v, [jj])
            pidx = zi * NSPEC + zj
            scb[pl.ds(k * 16, 16)] = plsc.load_gather(sp_v, [pidx])
            shb[pl.ds(k * 16, 16)] = plsc.load_gather(shp_v, [pidx])
        pltpu.sync_copy(scb, scl_h.at[pl.ds(base, CH_C)])
        pltpu.sync_copy(shb, shf_h.at[pl.ds(base, CH_C)])

    def valid(t):
        return w + t * STRIDE < nchunk

    @pl.when(valid(0))
    def _():
        start_idx(0, ivA, jvA, semIA)
        drain_idx(ivA, jvA, semIA)
        fire_g(ivA, jvA, gA0l, gA0r, gA1l, gA1r, semGA)

    @pl.when(valid(1))
    def _():
        start_idx(1, ivB, jvB, semIB)

    @pl.loop(0, itc, step=2)
    def _chunk(t):
        @pl.when(valid(t + 1))
        def _():
            drain_idx(ivB, jvB, semIB)
            fire_g(ivB, jvB, gB0l, gB0r, gB1l, gB1r, semGB)

        @pl.when(valid(t))
        def _():
            drain_g(ivA, gA0l, gA0r, gA1l, gA1r, semGA)
            compute(t, ivA, jvA, gA0l, gA0r, gA1l, gA1r)

        @pl.when(valid(t + 2))
        def _():
            start_idx(t + 2, ivA, jvA, semIA)
            drain_idx(ivA, jvA, semIA)
            fire_g(ivA, jvA, gA0l, gA0r, gA1l, gA1r, semGA)

        @pl.when(valid(t + 1))
        def _():
            drain_g(ivB, gB0l, gB0r, gB1l, gB1r, semGB)
            compute(t + 1, ivB, jvB, gB0l, gB0r, gB1l, gB1r)

        @pl.when(valid(t + 3))
        def _():
            start_idx(t + 3, ivB, jvB, semIB)


def _bc(an_pad, bci, bcj, acdl, acdr, sp_flat, shp_flat):
    ec = bci.shape[0]
    nchunk = ec // CH_C
    itc = -(-nchunk // (NC * NS))
    mesh = plsc.VectorSubcoreMesh(
        core_axis_name="c", subcore_axis_name="s", num_cores=NC, num_subcores=NS)
    f = pl.kernel(
        functools.partial(_bc_body, nchunk, itc),
        out_type=(
            jax.ShapeDtypeStruct((ec, HALF), F32),
            jax.ShapeDtypeStruct((ec, HALF), F32),
            jax.ShapeDtypeStruct((ec,), F32),
            jax.ShapeDtypeStruct((ec,), F32),
        ),
        mesh=mesh,
        compiler_params=pltpu.CompilerParams(needs_layout_passes=False),
        scratch_types=[
            pltpu.VMEM((N_PAD,), I32),
            pltpu.VMEM((NSPEC * NSPEC,), F32),
            pltpu.VMEM((NSPEC * NSPEC,), F32),
            pltpu.VMEM((CH_C,), I32),
            pltpu.VMEM((CH_C,), I32),
            pltpu.VMEM((CH_C,), I32),
            pltpu.VMEM((CH_C,), I32),
            pltpu.VMEM((CH_C, HALF), F32),
            pltpu.VMEM((CH_C, HALF), F32),
            pltpu.VMEM((CH_C, HALF), F32),
            pltpu.VMEM((CH_C, HALF), F32),
            pltpu.VMEM((CH_C, HALF), F32),
            pltpu.VMEM((CH_C, HALF), F32),
            pltpu.VMEM((CH_C, HALF), F32),
            pltpu.VMEM((CH_C, HALF), F32),
            pltpu.VMEM((CH_C,), F32),
            pltpu.VMEM((CH_C,), F32),
            pltpu.SemaphoreType.DMA,
            pltpu.SemaphoreType.DMA,
            pltpu.SemaphoreType.DMA,
            pltpu.SemaphoreType.DMA,
        ],
    )
    return f(an_pad, bci, bcj, acdl, acdr, sp_flat, shp_flat)


def _mlp(x, w1_ref, b1_ref, w2_ref, b2_ref, g_ref, b_ref):
    h = jnp.dot(x, w1_ref[...], preferred_element_type=F32) + b1_ref[...]
    h = h * jax.nn.sigmoid(h)
    o = jnp.dot(h, w2_ref[...], preferred_element_type=F32) + b2_ref[...]
    m = jnp.mean(o, axis=-1, keepdims=True)
    v = jnp.mean((o - m) * (o - m), axis=-1, keepdims=True)
    return (o - m) / jnp.sqrt(v + 1e-5) * g_ref[...] + b_ref[...]


def _off_body(bl_ref, br_ref, disp_ref, scl_ref, shf_ref, wrbf_ref,
              w1_ref, b1_ref, w2_ref, b2_ref, g_ref, b_ref, wo_ref, bo_ref,
              out_ref):
    w, cut = _radial(disp_ref[...], wrbf_ref)
    x = jnp.concatenate([bl_ref[...], br_ref[...]], axis=1) * (w * cut)
    o = _mlp(x, w1_ref, b1_ref, w2_ref, b2_ref, g_ref, b_ref)
    irr = jnp.dot(o, wo_ref[...], preferred_element_type=F32) + bo_ref[...]
    out_ref[...] = irr * scl_ref[...] + shf_ref[...] * cut


def _off(bl, br, disp_pad, scl, shf, wrbf, w1, b1, w2, b2, g, b, wo, bo):
    full = lambda r, c: pl.BlockSpec((r, c), lambda i: (0, 0))
    return pl.pallas_call(
        _off_body,
        grid=(bl.shape[0] // EBF,),
        in_specs=[
            pl.BlockSpec((EBF, HALF), lambda i: (i, 0)),
            pl.BlockSpec((EBF, HALF), lambda i: (i, 0)),
            pl.BlockSpec((EBF, 3), lambda i: (i, 0)),
            pl.BlockSpec((EBF, 1), lambda i: (i, 0)),
            pl.BlockSpec((EBF, 1), lambda i: (i, 0)),
            full(NRBF, D), full(D, H), full(1, H), full(H, D), full(1, D),
            full(1, D), full(1, D), full(D, DOFF), full(1, DOFF),
        ],
        out_specs=pl.BlockSpec((EBF, DOFF), lambda i: (i, 0)),
        out_shape=jax.ShapeDtypeStruct((bl.shape[0], DOFF), F32),
    )(bl, br, disp_pad, scl, shf, wrbf, w1, b1, w2, b2, g, b, wo, bo)


def _on_body(al_ref, ar_ref, son_ref, shon_ref,
             w1_ref, b1_ref, w2_ref, b2_ref, g_ref, b_ref, wo_ref, bo_ref,
             out_ref):
    x = jnp.concatenate([al_ref[...], ar_ref[...]], axis=1)
    o = _mlp(x, w1_ref, b1_ref, w2_ref, b2_ref, g_ref, b_ref)
    irr = jnp.dot(o, wo_ref[...], preferred_element_type=F32) + bo_ref[...]
    out_ref[...] = irr * son_ref[...] + shon_ref[...]


def _on(al, ar, son, shon, w1, b1, w2, b2, g, b, wo, bo):
    full = lambda r, c: pl.BlockSpec((r, c), lambda i: (0, 0))
    return pl.pallas_call(
        _on_body,
        grid=(N_PAD // EB,),
        in_specs=[
            pl.BlockSpec((EB, HALF), lambda i: (i, 0)),
            pl.BlockSpec((EB, HALF), lambda i: (i, 0)),
            pl.BlockSpec((EB, 1), lambda i: (i, 0)),
            pl.BlockSpec((EB, 1), lambda i: (i, 0)),
            full(D, H), full(1, H), full(H, D), full(1, D),
            full(1, D), full(1, D), full(D, DON), full(1, DON),
        ],
        out_specs=pl.BlockSpec((EB, DON), lambda i: (i, 0)),
        out_shape=jax.ShapeDtypeStruct((N_PAD, DON), F32),
    )(al, ar, son, shon, w1, b1, w2, b2, g, b, wo, bo)


def kernel(atomic_numbers, bc_neighbour_indices, bc_neighbour_displacements,
           ac_neighbour_indices, ac_neighbour_displacements, Z_table,
           W_rbf_ac, W_rbf_bc, W1, b1, W2, b2, ln1_g, ln1_b, ln2_g, ln2_b,
           W_off, b_off, W_on, b_on, scale_pair, shift_pair, scale_on,
           shift_on):
    an = atomic_numbers.astype(I32)
    an_pad = jnp.pad(an, (0, N_PAD - N))
    aci = ac_neighbour_indices[:, 0].astype(I32)
    acj = ac_neighbour_indices[:, 1].astype(I32)
    bci = bc_neighbour_indices[:, 0].astype(I32)
    bcj = bc_neighbour_indices[:, 1].astype(I32)
    zl = Z_table[:, :HALF]
    zr = Z_table[:, HALF:]
    zpad = jnp.pad(Z_table, ((0, 128 - NSPEC), (0, 0)))
    sont = jnp.pad(scale_on, (0, 128 - NSPEC))
    shont = jnp.pad(shift_on, (0, 128 - NSPEC))

    zj = _zjk(an_pad, acj)
    ml, mr = _msg(ac_neighbour_displacements, zj.reshape(E, 1), W_rbf_ac, zpad)
    acdl, acdr, son, shon = _acd(an_pad, aci, zl, zr, ml, mr, sont, shont)

    on = _on(acdl, acdr, son.reshape(N_PAD, 1), shon.reshape(N_PAD, 1),
             W1, b1.reshape(1, H), W2, b2.reshape(1, D),
             ln2_g.reshape(1, D), ln2_b.reshape(1, D), W_on,
             b_on.reshape(1, DON))

    sp_flat = scale_pair.reshape(-1)
    shp_flat = shift_pair.reshape(-1)
    ec = E // NSPLIT
    offs = []
    for k in range(NSPLIT):
        sl = slice(k * ec, (k + 1) * ec)
        bl, br, scl, shf = _bc(an_pad, bci[sl], bcj[sl], acdl, acdr,
                               sp_flat, shp_flat)
        offs.append(_off(bl, br, bc_neighbour_displacements[sl],
                         scl.reshape(ec, 1), shf.reshape(ec, 1),
                         W_rbf_bc, W1, b1.reshape(1, H), W2, b2.reshape(1, D),
                         ln1_g.reshape(1, D), ln1_b.reshape(1, D), W_off,
                         b_off.reshape(1, DOFF)))
    off = jnp.concatenate(offs, axis=0) if NSPLIT > 1 else offs[0]
    return off, on[:N]


# speedup vs baseline: 1.0211x; 1.0211x over previous
import functools

import jax
import jax.numpy as jnp
from jax import lax
from jax.experimental import pallas as pl
from jax.experimental.pallas import tpu as pltpu
from jax.experimental.pallas import tpu_sc as plsc

N = 10000
E = 160000
D = 256
HALF = 128
H = 512
NRBF = 16
NSPEC = 100
DOFF = 64
DON = 64
CUTOFF = 5.0
F32 = jnp.float32
I32 = jnp.int32

NC, NS = 2, 16
N_PAD = 10240
NODES_PER_TILE = N_PAD // NS
E_PER_TILE_A = E // NS
CH_A = 80
IT_A = E_PER_TILE_A // CH_A
CH_C = 64
NCHUNK_C = E // CH_C
IT_C = -(-NCHUNK_C // (NC * NS))
EB = 512
EBF = 800
NSPLIT = 1


def _axis(name):
    return lax.axis_index(name)


def _gather_rows(tbl_h, idx_ref, dst, sem):
    pltpu.async_copy(tbl_h.at[idx_ref], dst, sem).wait()


def _gather_start(tbl_h, idx_ref, dst, sem):
    return pltpu.async_copy(tbl_h.at[idx_ref], dst, sem)


def _scatter_add_rows(src, acc, idx_ref):
    pltpu.sync_copy(src, acc.at[idx_ref], add=True)


def _cutoff_fn(r):
    return jnp.where(r < CUTOFF, 0.5 * (jnp.cos(jnp.pi * r / CUTOFF) + 1.0), 0.0)


def _radial(disp, wrbf_ref):
    r = jnp.sqrt(jnp.sum(disp * disp, axis=1, keepdims=True))
    mu = lax.broadcasted_iota(I32, (1, NRBF), 1).astype(F32) * (CUTOFF / (NRBF - 1))
    phi = jnp.exp(-10.0 * (r - mu) ** 2)
    cut = _cutoff_fn(r)
    w = jnp.dot(phi, wrbf_ref[...], preferred_element_type=F32)
    return w, cut


EPW = E // (NC * NS)
EPW_PAD = EPW + 8


def _zjk_body(an_h, acj_h, zj_h, an_v, jv, zb):
    c = _axis("c")
    s = _axis("s")
    w = s * NC + c
    pltpu.sync_copy(an_h, an_v)
    base = w * EPW
    pltpu.sync_copy(acj_h.at[pl.ds(base, EPW)], jv.at[pl.ds(0, EPW)])
    lane = lax.iota(I32, 16)

    @pl.loop(0, EPW_PAD // 16)
    def _(k):
        off = k * 16
        jj = jv[pl.ds(off, 16)]
        jj = jnp.where(lane < EPW - off, jj, 0)
        zb[pl.ds(off, 16)] = plsc.load_gather(an_v, [jj])

    pltpu.sync_copy(zb.at[pl.ds(0, EPW)], zj_h.at[pl.ds(base, EPW)])


def _zjk(an_pad, acj):
    mesh = plsc.VectorSubcoreMesh(
        core_axis_name="c", subcore_axis_name="s", num_cores=NC, num_subcores=NS)
    f = pl.kernel(
        _zjk_body,
        out_type=jax.ShapeDtypeStruct((E,), I32),
        mesh=mesh,
        compiler_params=pltpu.CompilerParams(needs_layout_passes=False),
        scratch_types=[
            pltpu.VMEM((N_PAD,), I32),
            pltpu.VMEM((EPW_PAD,), I32),
            pltpu.VMEM((EPW_PAD,), I32),
        ],
    )
    return f(an_pad, acj)


WB = 640


def _msg_body(disp_ref, zj_ref, wrbf_ref, zpad_ref, out_l_ref, out_r_ref):
    w, cut = _radial(disp_ref[...], wrbf_ref)
    z = zj_ref[...]
    oh = (z == lax.broadcasted_iota(I32, (1, 128), 1)).astype(F32)
    zq = zpad_ref[...]
    hi = lax.bitcast_convert_type(
        lax.bitcast_convert_type(zq, I32) & jnp.int32(-65536), F32)
    lo = zq - hi
    wc = w * cut
    m = (wc * jnp.dot(oh, hi, preferred_element_type=F32)
         + wc * jnp.dot(oh, lo, preferred_element_type=F32))
    out_l_ref[...] = m[:, :HALF]
    out_r_ref[...] = m[:, HALF:]


def _msg(disp, zj2, wrbf, zpad):
    return pl.pallas_call(
        _msg_body,
        grid=(E // WB,),
        in_specs=[
            pl.BlockSpec((WB, 3), lambda i: (i, 0)),
            pl.BlockSpec((WB, 1), lambda i: (i, 0)),
            pl.BlockSpec((NRBF, D), lambda i: (0, 0)),
            pl.BlockSpec((128, D), lambda i: (0, 0)),
        ],
        out_specs=[pl.BlockSpec((WB, HALF), lambda i: (i, 0))] * 2,
        out_shape=[jax.ShapeDtypeStruct((E, HALF), F32)] * 2,
    )(disp, zj2, wrbf, zpad)


def _acd_body(an_h, aci_h, zl_h, zr_h, ml_h, mr_h, sont_h, shont_h,
              acdl_h, acdr_h, son_h, shon_h,
              an_v, sont_v, shont_v, iv0, iv1, iv2, m0, m1, m2,
              sbuf, shbuf, acc, sem0, sem1, sem2):
    c = _axis("c")
    s = _axis("s")
    node_base = s * NODES_PER_TILE

    def half(z_h, m_h, acd_h):
        for k in range(NODES_PER_TILE // CH_A):
            off = node_base + k * CH_A
            pltpu.sync_copy(an_h.at[pl.ds(off, CH_A)], iv0)
            _gather_rows(z_h, iv0, m0, sem0)
            pltpu.sync_copy(m0, acc.at[pl.ds(off, CH_A)])
        plsc.subcore_barrier()
        ebase = s * E_PER_TILE_A

        def start(t, ivb, mb, sm):
            base = ebase + t * CH_A
            pltpu.async_copy(aci_h.at[pl.ds(base, CH_A)], ivb, sm)
            pltpu.async_copy(m_h.at[pl.ds(base, CH_A)], mb, sm)

        def drain(ivb, mb, sm):
            pltpu.make_async_copy(aci_h.at[pl.ds(0, CH_A)], ivb, sm).wait()
            pltpu.make_async_copy(m_h.at[pl.ds(0, CH_A)], mb, sm).wait()

        start(0, iv0, m0, sem0)
        start(1, iv1, m1, sem1)

        @pl.loop(0, IT_A, step=3)
        def _edge_chunk(t):
            drain(iv0, m0, sem0)
            _scatter_add_rows(m0, acc, iv0)

            @pl.when(t + 2 < IT_A)
            def _():
                start(t + 2, iv2, m2, sem2)

            @pl.when(t + 1 < IT_A)
            def _():
                drain(iv1, m1, sem1)
                _scatter_add_rows(m1, acc, iv1)

                @pl.when(t + 3 < IT_A)
                def _():
                    start(t + 3, iv0, m0, sem0)

            @pl.when(t + 2 < IT_A)
            def _():
                drain(iv2, m2, sem2)
                _scatter_add_rows(m2, acc, iv2)

                @pl.when(t + 4 < IT_A)
                def _():
                    start(t + 4, iv1, m1, sem1)

        plsc.subcore_barrier()
        pltpu.sync_copy(acc.at[pl.ds(node_base, NODES_PER_TILE)],
                        acd_h.at[pl.ds(node_base, NODES_PER_TILE)])

    @pl.when(c == 0)
    def _():
        half(zl_h, ml_h, acdl_h)
        pltpu.sync_copy(an_h, an_v)
        pltpu.sync_copy(sont_h, sont_v)
        pltpu.sync_copy(shont_h, shont_v)

        @pl.loop(0, NODES_PER_TILE // 16)
        def _n16(k):
            zn = an_v[pl.ds(node_base + k * 16, 16)]
            sbuf[pl.ds(k * 16, 16)] = plsc.load_gather(sont_v, [zn])
            shbuf[pl.ds(k * 16, 16)] = plsc.load_gather(shont_v, [zn])

        pltpu.sync_copy(sbuf, son_h.at[pl.ds(node_base, NODES_PER_TILE)])
        pltpu.sync_copy(shbuf, shon_h.at[pl.ds(node_base, NODES_PER_TILE)])

    @pl.when(c == 1)
    def _():
        half(zr_h, mr_h, acdr_h)


def _acd(an_pad, aci, zl, zr, ml, mr, sont, shont):
    mesh = plsc.VectorSubcoreMesh(
        core_axis_name="c", subcore_axis_name="s", num_cores=NC, num_subcores=NS)
    f = pl.kernel(
        _acd_body,
        out_type=(
            jax.ShapeDtypeStruct((N_PAD, HALF), F32),
            jax.ShapeDtypeStruct((N_PAD, HALF), F32),
            jax.ShapeDtypeStruct((N_PAD,), F32),
            jax.ShapeDtypeStruct((N_PAD,), F32),
        ),
        mesh=mesh,
        compiler_params=pltpu.CompilerParams(needs_layout_passes=False),
        scratch_types=[
            pltpu.VMEM((N_PAD,), I32),
            pltpu.VMEM((128,), F32),
            pltpu.VMEM((128,), F32),
            pltpu.VMEM((CH_A,), I32),
            pltpu.VMEM((CH_A,), I32),
            pltpu.VMEM((CH_A,), I32),
            pltpu.VMEM((CH_A, HALF), F32),
            pltpu.VMEM((CH_A, HALF), F32),
            pltpu.VMEM((CH_A, HALF), F32),
            pltpu.VMEM((NODES_PER_TILE,), F32),
            pltpu.VMEM((NODES_PER_TILE,), F32),
            pltpu.VMEM_SHARED((N_PAD, HALF), F32),
            pltpu.SemaphoreType.DMA,
            pltpu.SemaphoreType.DMA,
            pltpu.SemaphoreType.DMA,
        ],
    )
    return f(an_pad, aci, zl, zr, ml, mr, sont, shont)


def _bc_body(nchunk, itc, an_h, bci_h, bcj_h, acdl_h, acdr_h, sp_h, shp_h,
             bl_h, br_h, scl_h, shf_h,
             an_v, sp_v, shp_v, ivA, jvA, ivB, jvB,
             gA0l, gA0r, gA1l, gA1r, gB0l, gB0r, gB1l, gB1r,
             scb, shb, semIA, semIB, semGA, semGB):
    c = _axis("c")
    s = _axis("s")
    w = s * NC + c
    pltpu.sync_copy(an_h, an_v)
    pltpu.sync_copy(sp_h, sp_v)
    pltpu.sync_copy(shp_h, shp_v)
    STRIDE = NC * NS

    def start_idx(t, ivb, jvb, sm):
        base = (w + t * STRIDE) * CH_C
        pltpu.async_copy(bci_h.at[pl.ds(base, CH_C)], ivb, sm)
        pltpu.async_copy(bcj_h.at[pl.ds(base, CH_C)], jvb, sm)

    def drain_idx(ivb, jvb, sm):
        pltpu.make_async_copy(bci_h.at[pl.ds(0, CH_C)], ivb, sm).wait()
        pltpu.make_async_copy(bcj_h.at[pl.ds(0, CH_C)], jvb, sm).wait()

    def fire_g(ivb, jvb, g0lb, g0rb, g1lb, g1rb, sm):
        _gather_start(acdl_h, ivb, g0lb, sm)
        _gather_start(acdr_h, ivb, g0rb, sm)
        _gather_start(acdl_h, jvb, g1lb, sm)
        _gather_start(acdr_h, jvb, g1rb, sm)

    def drain_g(ivb, g0lb, g0rb, g1lb, g1rb, sm):
        pltpu.make_async_copy(acdl_h.at[ivb], g0lb, sm).wait()
        pltpu.make_async_copy(acdr_h.at[ivb], g0rb, sm).wait()
        pltpu.make_async_copy(acdl_h.at[ivb], g1lb, sm).wait()
        pltpu.make_async_copy(acdr_h.at[ivb], g1rb, sm).wait()

    def compute(t, ivb, jvb, g0lb, g0rb, g1lb, g1rb):
        base = (w + t * STRIDE) * CH_C

        @pl.loop(0, CH_C)
        def _row(rr):
            for c0 in range(0, HALF, 16):
                g0lb[rr, pl.ds(c0, 16)] = (
                    g0lb[rr, pl.ds(c0, 16)] * g1lb[rr, pl.ds(c0, 16)])
                g0rb[rr, pl.ds(c0, 16)] = (
                    g0rb[rr, pl.ds(c0, 16)] * g1rb[rr, pl.ds(c0, 16)])

        pltpu.sync_copy(g0lb, bl_h.at[pl.ds(base, CH_C)])
        pltpu.sync_copy(g0rb, br_h.at[pl.ds(base, CH_C)])
        for k in range(CH_C // 16):
            ii = ivb[pl.ds(k * 16, 16)]
            jj = jvb[pl.ds(k * 16, 16)]
            zi = plsc.load_gather(an_v, [ii])
            zj = plsc.load_gather(an_v, [jj])
            pidx = zi * NSPEC + zj
            scb[pl.ds(k * 16, 16)] = plsc.load_gather(sp_v, [pidx])
            shb[pl.ds(k * 16, 16)] = plsc.load_gather(shp_v, [pidx])
        pltpu.sync_copy(scb, scl_h.at[pl.ds(base, CH_C)])
        pltpu.sync_copy(shb, shf_h.at[pl.ds(base, CH_C)])

    def valid(t):
        return w + t * STRIDE < nchunk

    @pl.when(valid(0))
    def _():
        start_idx(0, ivA, jvA, semIA)
        drain_idx(ivA, jvA, semIA)
        fire_g(ivA, jvA, gA0l, gA0r, gA1l, gA1r, semGA)

    @pl.when(valid(1))
    def _():
        start_idx(1, ivB, jvB, semIB)

    @pl.loop(0, itc, step=2)
    def _chunk(t):
        @pl.when(valid(t + 1))
        def _():
            drain_idx(ivB, jvB, semIB)
            fire_g(ivB, jvB, gB0l, gB0r, gB1l, gB1r, semGB)

        @pl.when(valid(t))
        def _():
            drain_g(ivA, gA0l, gA0r, gA1l, gA1r, semGA)
            compute(t, ivA, jvA, gA0l, gA0r, gA1l, gA1r)

        @pl.when(valid(t + 2))
        def _():
            start_idx(t + 2, ivA, jvA, semIA)
            drain_idx(ivA, jvA, semIA)
            fire_g(ivA, jvA, gA0l, gA0r, gA1l, gA1r, semGA)

        @pl.when(valid(t + 1))
        def _():
            drain_g(ivB, gB0l, gB0r, gB1l, gB1r, semGB)
            compute(t + 1, ivB, jvB, gB0l, gB0r, gB1l, gB1r)

        @pl.when(valid(t + 3))
        def _():
            start_idx(t + 3, ivB, jvB, semIB)


def _bc(an_pad, bci, bcj, acdl, acdr, sp_flat, shp_flat):
    ec = bci.shape[0]
    nchunk = ec // CH_C
    itc = -(-nchunk // (NC * NS))
    mesh = plsc.VectorSubcoreMesh(
        core_axis_name="c", subcore_axis_name="s", num_cores=NC, num_subcores=NS)
    f = pl.kernel(
        functools.partial(_bc_body, nchunk, itc),
        out_type=(
            jax.ShapeDtypeStruct((ec, HALF), F32),
            jax.ShapeDtypeStruct((ec, HALF), F32),
            jax.ShapeDtypeStruct((ec,), F32),
            jax.ShapeDtypeStruct((ec,), F32),
        ),
        mesh=mesh,
        compiler_params=pltpu.CompilerParams(needs_layout_passes=False),
        scratch_types=[
            pltpu.VMEM((N_PAD,), I32),
            pltpu.VMEM((NSPEC * NSPEC,), F32),
            pltpu.VMEM((NSPEC * NSPEC,), F32),
            pltpu.VMEM((CH_C,), I32),
            pltpu.VMEM((CH_C,), I32),
            pltpu.VMEM((CH_C,), I32),
            pltpu.VMEM((CH_C,), I32),
            pltpu.VMEM((CH_C, HALF), F32),
            pltpu.VMEM((CH_C, HALF), F32),
            pltpu.VMEM((CH_C, HALF), F32),
            pltpu.VMEM((CH_C, HALF), F32),
            pltpu.VMEM((CH_C, HALF), F32),
            pltpu.VMEM((CH_C, HALF), F32),
            pltpu.VMEM((CH_C, HALF), F32),
            pltpu.VMEM((CH_C, HALF), F32),
            pltpu.VMEM((CH_C,), F32),
            pltpu.VMEM((CH_C,), F32),
            pltpu.SemaphoreType.DMA,
            pltpu.SemaphoreType.DMA,
            pltpu.SemaphoreType.DMA,
            pltpu.SemaphoreType.DMA,
        ],
    )
    return f(an_pad, bci, bcj, acdl, acdr, sp_flat, shp_flat)


def _mlp(x, w1_ref, b1_ref, w2_ref, b2_ref, g_ref, b_ref):
    h = jnp.dot(x, w1_ref[...], preferred_element_type=F32) + b1_ref[...]
    h = h * jax.nn.sigmoid(h)
    o = jnp.dot(h, w2_ref[...], preferred_element_type=F32) + b2_ref[...]
    m = jnp.mean(o, axis=-1, keepdims=True)
    v = jnp.mean((o - m) * (o - m), axis=-1, keepdims=True)
    return (o - m) / jnp.sqrt(v + 1e-5) * g_ref[...] + b_ref[...]


def _off_body(bl_ref, br_ref, disp_ref, scl_ref, shf_ref, wrbf_ref,
              w1_ref, b1_ref, w2_ref, b2_ref, g_ref, b_ref, wo_ref, bo_ref,
              out_ref):
    w, cut = _radial(disp_ref[...], wrbf_ref)
    x = jnp.concatenate([bl_ref[...], br_ref[...]], axis=1) * (w * cut)
    o = _mlp(x, w1_ref, b1_ref, w2_ref, b2_ref, g_ref, b_ref)
    irr = jnp.dot(o, wo_ref[...], preferred_element_type=F32) + bo_ref[...]
    out_ref[...] = irr * scl_ref[...] + shf_ref[...] * cut


def _off(bl, br, disp_pad, scl, shf, wrbf, w1, b1, w2, b2, g, b, wo, bo):
    full = lambda r, c: pl.BlockSpec((r, c), lambda i: (0, 0))
    return pl.pallas_call(
        _off_body,
        grid=(bl.shape[0] // EBF,),
        in_specs=[
            pl.BlockSpec((EBF, HALF), lambda i: (i, 0)),
            pl.BlockSpec((EBF, HALF), lambda i: (i, 0)),
            pl.BlockSpec((EBF, 3), lambda i: (i, 0)),
            pl.BlockSpec((EBF, 1), lambda i: (i, 0)),
            pl.BlockSpec((EBF, 1), lambda i: (i, 0)),
            full(NRBF, D), full(D, H), full(1, H), full(H, D), full(1, D),
            full(1, D), full(1, D), full(D, DOFF), full(1, DOFF),
        ],
        out_specs=pl.BlockSpec((EBF, DOFF), lambda i: (i, 0)),
        out_shape=jax.ShapeDtypeStruct((bl.shape[0], DOFF), F32),
    )(bl, br, disp_pad, scl, shf, wrbf, w1, b1, w2, b2, g, b, wo, bo)


def _on_body(al_ref, ar_ref, son_ref, shon_ref,
             w1_ref, b1_ref, w2_ref, b2_ref, g_ref, b_ref, wo_ref, bo_ref,
             out_ref):
    x = jnp.concatenate([al_ref[...], ar_ref[...]], axis=1)
    o = _mlp(x, w1_ref, b1_ref, w2_ref, b2_ref, g_ref, b_ref)
    irr = jnp.dot(o, wo_ref[...], preferred_element_type=F32) + bo_ref[...]
    out_ref[...] = irr * son_ref[...] + shon_ref[...]


def _on(al, ar, son, shon, w1, b1, w2, b2, g, b, wo, bo):
    full = lambda r, c: pl.BlockSpec((r, c), lambda i: (0, 0))
    return pl.pallas_call(
        _on_body,
        grid=(N_PAD // EB,),
        in_specs=[
            pl.BlockSpec((EB, HALF), lambda i: (i, 0)),
            pl.BlockSpec((EB, HALF), lambda i: (i, 0)),
            pl.BlockSpec((EB, 1), lambda i: (i, 0)),
            pl.BlockSpec((EB, 1), lambda i: (i, 0)),
            full(D, H), full(1, H), full(H, D), full(1, D),
            full(1, D), full(1, D), full(D, DON), full(1, DON),
        ],
        out_specs=pl.BlockSpec((EB, DON), lambda i: (i, 0)),
        out_shape=jax.ShapeDtypeStruct((N_PAD, DON), F32),
    )(al, ar, son, shon, w1, b1, w2, b2, g, b, wo, bo)


def kernel(atomic_numbers, bc_neighbour_indices, bc_neighbour_displacements,
           ac_neighbour_indices, ac_neighbour_displacements, Z_table,
           W_rbf_ac, W_rbf_bc, W1, b1, W2, b2, ln1_g, ln1_b, ln2_g, ln2_b,
           W_off, b_off, W_on, b_on, scale_pair, shift_pair, scale_on,
           shift_on):
    an = atomic_numbers.astype(I32)
    an_pad = jnp.pad(an, (0, N_PAD - N))
    aci = ac_neighbour_indices[:, 0].astype(I32)
    acj = ac_neighbour_indices[:, 1].astype(I32)
    bci = bc_neighbour_indices[:, 0].astype(I32)
    bcj = bc_neighbour_indices[:, 1].astype(I32)
    zl = Z_table[:, :HALF]
    zr = Z_table[:, HALF:]
    zpad = jnp.pad(Z_table, ((0, 128 - NSPEC), (0, 0)))
    sont = jnp.pad(scale_on, (0, 128 - NSPEC))
    shont = jnp.pad(shift_on, (0, 128 - NSPEC))

    zj = _zjk(an_pad, acj)
    ml, mr = _msg(ac_neighbour_displacements, zj.reshape(E, 1), W_rbf_ac, zpad)
    acdl, acdr, son, shon = _acd(an_pad, aci, zl, zr, ml, mr, sont, shont)

    on = _on(acdl, acdr, son.reshape(N_PAD, 1), shon.reshape(N_PAD, 1),
             W1, b1.reshape(1, H), W2, b2.reshape(1, D),
             ln2_g.reshape(1, D), ln2_b.reshape(1, D), W_on,
             b_on.reshape(1, DON))

    sp_flat = scale_pair.reshape(-1)
    shp_flat = shift_pair.reshape(-1)
    ec = E // NSPLIT
    offs = []
    for k in range(NSPLIT):
        sl = slice(k * ec, (k + 1) * ec)
        bl, br, scl, shf = _bc(an_pad, bci[sl], bcj[sl], acdl, acdr,
                               sp_flat, shp_flat)
        offs.append(_off(bl, br, bc_neighbour_displacements[sl],
                         scl.reshape(ec, 1), shf.reshape(ec, 1),
                         W_rbf_bc, W1, b1.reshape(1, H), W2, b2.reshape(1, D),
                         ln1_g.reshape(1, D), ln1_b.reshape(1, D), W_off,
                         b_off.reshape(1, DOFF)))
    off = jnp.concatenate(offs, axis=0) if NSPLIT > 1 else offs[0]
    return off, on[:N]


# speedup vs baseline: 1.0479x; 1.0263x over previous
import functools

import jax
import jax.numpy as jnp
from jax import lax
from jax.experimental import pallas as pl
from jax.experimental.pallas import tpu as pltpu
from jax.experimental.pallas import tpu_sc as plsc

N = 10000
E = 160000
D = 256
HALF = 128
H = 512
NRBF = 16
NSPEC = 100
DOFF = 64
DON = 64
CUTOFF = 5.0
F32 = jnp.float32
I32 = jnp.int32

NC, NS = 2, 16
N_PAD = 10240
NODES_PER_TILE = N_PAD // NS
E_PER_TILE_A = E // NS
CH_A = 80
IT_A = E_PER_TILE_A // CH_A
CH_C = 64
NCHUNK_C = E // CH_C
IT_C = -(-NCHUNK_C // (NC * NS))
EB = 512
EBF = 1000
NSPLIT = 1


def _axis(name):
    return lax.axis_index(name)


def _gather_rows(tbl_h, idx_ref, dst, sem):
    pltpu.async_copy(tbl_h.at[idx_ref], dst, sem).wait()


def _gather_start(tbl_h, idx_ref, dst, sem):
    return pltpu.async_copy(tbl_h.at[idx_ref], dst, sem)


def _scatter_add_rows(src, acc, idx_ref):
    pltpu.sync_copy(src, acc.at[idx_ref], add=True)


def _cutoff_fn(r):
    return jnp.where(r < CUTOFF, 0.5 * (jnp.cos(jnp.pi * r / CUTOFF) + 1.0), 0.0)


def _radial(disp, wrbf_ref):
    r = jnp.sqrt(jnp.sum(disp * disp, axis=1, keepdims=True))
    mu = lax.broadcasted_iota(I32, (1, NRBF), 1).astype(F32) * (CUTOFF / (NRBF - 1))
    phi = jnp.exp(-10.0 * (r - mu) ** 2)
    cut = _cutoff_fn(r)
    w = jnp.dot(phi, wrbf_ref[...], preferred_element_type=F32)
    return w, cut


EPW = E // (NC * NS)
EPW_PAD = EPW + 8


def _zjk_body(an_h, acj_h, zj_h, an_v, jv, zb):
    c = _axis("c")
    s = _axis("s")
    w = s * NC + c
    pltpu.sync_copy(an_h, an_v)
    base = w * EPW
    pltpu.sync_copy(acj_h.at[pl.ds(base, EPW)], jv.at[pl.ds(0, EPW)])
    lane = lax.iota(I32, 16)

    @pl.loop(0, EPW_PAD // 16)
    def _(k):
        off = k * 16
        jj = jv[pl.ds(off, 16)]
        jj = jnp.where(lane < EPW - off, jj, 0)
        zb[pl.ds(off, 16)] = plsc.load_gather(an_v, [jj])

    pltpu.sync_copy(zb.at[pl.ds(0, EPW)], zj_h.at[pl.ds(base, EPW)])


def _zjk(an_pad, acj):
    mesh = plsc.VectorSubcoreMesh(
        core_axis_name="c", subcore_axis_name="s", num_cores=NC, num_subcores=NS)
    f = pl.kernel(
        _zjk_body,
        out_type=jax.ShapeDtypeStruct((E,), I32),
        mesh=mesh,
        compiler_params=pltpu.CompilerParams(needs_layout_passes=False),
        scratch_types=[
            pltpu.VMEM((N_PAD,), I32),
            pltpu.VMEM((EPW_PAD,), I32),
            pltpu.VMEM((EPW_PAD,), I32),
        ],
    )
    return f(an_pad, acj)


WB = 1000


def _msg_body(disp_ref, zj_ref, wrbf_ref, zpad_ref, out_l_ref, out_r_ref):
    w, cut = _radial(disp_ref[...], wrbf_ref)
    z = zj_ref[...]
    oh = (z == lax.broadcasted_iota(I32, (1, 128), 1)).astype(F32)
    zq = zpad_ref[...]
    hi = lax.bitcast_convert_type(
        lax.bitcast_convert_type(zq, I32) & jnp.int32(-65536), F32)
    lo = zq - hi
    wc = w * cut
    m = (wc * jnp.dot(oh, hi, preferred_element_type=F32)
         + wc * jnp.dot(oh, lo, preferred_element_type=F32))
    out_l_ref[...] = m[:, :HALF]
    out_r_ref[...] = m[:, HALF:]


def _msg(disp, zj2, wrbf, zpad):
    return pl.pallas_call(
        _msg_body,
        grid=(E // WB,),
        in_specs=[
            pl.BlockSpec((WB, 3), lambda i: (i, 0)),
            pl.BlockSpec((WB, 1), lambda i: (i, 0)),
            pl.BlockSpec((NRBF, D), lambda i: (0, 0)),
            pl.BlockSpec((128, D), lambda i: (0, 0)),
        ],
        out_specs=[pl.BlockSpec((WB, HALF), lambda i: (i, 0))] * 2,
        out_shape=[jax.ShapeDtypeStruct((E, HALF), F32)] * 2,
    )(disp, zj2, wrbf, zpad)


def _acd_body(an_h, aci_h, zl_h, zr_h, ml_h, mr_h, sont_h, shont_h,
              acdl_h, acdr_h, son_h, shon_h,
              an_v, sont_v, shont_v, iv0, iv1, iv2, m0, m1, m2,
              sbuf, shbuf, acc, sem0, sem1, sem2):
    c = _axis("c")
    s = _axis("s")
    node_base = s * NODES_PER_TILE

    def half(z_h, m_h, acd_h):
        for k in range(NODES_PER_TILE // CH_A):
            off = node_base + k * CH_A
            pltpu.sync_copy(an_h.at[pl.ds(off, CH_A)], iv0)
            _gather_rows(z_h, iv0, m0, sem0)
            pltpu.sync_copy(m0, acc.at[pl.ds(off, CH_A)])
        plsc.subcore_barrier()
        ebase = s * E_PER_TILE_A

        def start(t, ivb, mb, sm):
            base = ebase + t * CH_A
            pltpu.async_copy(aci_h.at[pl.ds(base, CH_A)], ivb, sm)
            pltpu.async_copy(m_h.at[pl.ds(base, CH_A)], mb, sm)

        def drain(ivb, mb, sm):
            pltpu.make_async_copy(aci_h.at[pl.ds(0, CH_A)], ivb, sm).wait()
            pltpu.make_async_copy(m_h.at[pl.ds(0, CH_A)], mb, sm).wait()

        start(0, iv0, m0, sem0)
        start(1, iv1, m1, sem1)

        @pl.loop(0, IT_A, step=3)
        def _edge_chunk(t):
            drain(iv0, m0, sem0)
            _scatter_add_rows(m0, acc, iv0)

            @pl.when(t + 2 < IT_A)
            def _():
                start(t + 2, iv2, m2, sem2)

            @pl.when(t + 1 < IT_A)
            def _():
                drain(iv1, m1, sem1)
                _scatter_add_rows(m1, acc, iv1)

                @pl.when(t + 3 < IT_A)
                def _():
                    start(t + 3, iv0, m0, sem0)

            @pl.when(t + 2 < IT_A)
            def _():
                drain(iv2, m2, sem2)
                _scatter_add_rows(m2, acc, iv2)

                @pl.when(t + 4 < IT_A)
                def _():
                    start(t + 4, iv1, m1, sem1)

        plsc.subcore_barrier()
        pltpu.sync_copy(acc.at[pl.ds(node_base, NODES_PER_TILE)],
                        acd_h.at[pl.ds(node_base, NODES_PER_TILE)])

    @pl.when(c == 0)
    def _():
        half(zl_h, ml_h, acdl_h)
        pltpu.sync_copy(an_h, an_v)
        pltpu.sync_copy(sont_h, sont_v)
        pltpu.sync_copy(shont_h, shont_v)

        @pl.loop(0, NODES_PER_TILE // 16)
        def _n16(k):
            zn = an_v[pl.ds(node_base + k * 16, 16)]
            sbuf[pl.ds(k * 16, 16)] = plsc.load_gather(sont_v, [zn])
            shbuf[pl.ds(k * 16, 16)] = plsc.load_gather(shont_v, [zn])

        pltpu.sync_copy(sbuf, son_h.at[pl.ds(node_base, NODES_PER_TILE)])
        pltpu.sync_copy(shbuf, shon_h.at[pl.ds(node_base, NODES_PER_TILE)])

    @pl.when(c == 1)
    def _():
        half(zr_h, mr_h, acdr_h)


def _acd(an_pad, aci, zl, zr, ml, mr, sont, shont):
    mesh = plsc.VectorSubcoreMesh(
        core_axis_name="c", subcore_axis_name="s", num_cores=NC, num_subcores=NS)
    f = pl.kernel(
        _acd_body,
        out_type=(
            jax.ShapeDtypeStruct((N_PAD, HALF), F32),
            jax.ShapeDtypeStruct((N_PAD, HALF), F32),
            jax.ShapeDtypeStruct((N_PAD,), F32),
            jax.ShapeDtypeStruct((N_PAD,), F32),
        ),
        mesh=mesh,
        compiler_params=pltpu.CompilerParams(needs_layout_passes=False),
        scratch_types=[
            pltpu.VMEM((N_PAD,), I32),
            pltpu.VMEM((128,), F32),
            pltpu.VMEM((128,), F32),
            pltpu.VMEM((CH_A,), I32),
            pltpu.VMEM((CH_A,), I32),
            pltpu.VMEM((CH_A,), I32),
            pltpu.VMEM((CH_A, HALF), F32),
            pltpu.VMEM((CH_A, HALF), F32),
            pltpu.VMEM((CH_A, HALF), F32),
            pltpu.VMEM((NODES_PER_TILE,), F32),
            pltpu.VMEM((NODES_PER_TILE,), F32),
            pltpu.VMEM_SHARED((N_PAD, HALF), F32),
            pltpu.SemaphoreType.DMA,
            pltpu.SemaphoreType.DMA,
            pltpu.SemaphoreType.DMA,
        ],
    )
    return f(an_pad, aci, zl, zr, ml, mr, sont, shont)


def _bc_body(nchunk, itc, an_h, bci_h, bcj_h, acdl_h, acdr_h, sp_h, shp_h,
             bl_h, br_h, scl_h, shf_h,
             an_v, sp_v, shp_v, ivA, jvA, ivB, jvB,
             gA0l, gA0r, gA1l, gA1r, gB0l, gB0r, gB1l, gB1r,
             scb, shb, semIA, semIB, semGA, semGB):
    c = _axis("c")
    s = _axis("s")
    w = s * NC + c
    pltpu.sync_copy(an_h, an_v)
    pltpu.sync_copy(sp_h, sp_v)
    pltpu.sync_copy(shp_h, shp_v)
    STRIDE = NC * NS

    def start_idx(t, ivb, jvb, sm):
        base = (w + t * STRIDE) * CH_C
        pltpu.async_copy(bci_h.at[pl.ds(base, CH_C)], ivb, sm)
        pltpu.async_copy(bcj_h.at[pl.ds(base, CH_C)], jvb, sm)

    def drain_idx(ivb, jvb, sm):
        pltpu.make_async_copy(bci_h.at[pl.ds(0, CH_C)], ivb, sm).wait()
        pltpu.make_async_copy(bcj_h.at[pl.ds(0, CH_C)], jvb, sm).wait()

    def fire_g(ivb, jvb, g0lb, g0rb, g1lb, g1rb, sm):
        _gather_start(acdl_h, ivb, g0lb, sm)
        _gather_start(acdr_h, ivb, g0rb, sm)
        _gather_start(acdl_h, jvb, g1lb, sm)
        _gather_start(acdr_h, jvb, g1rb, sm)

    def drain_g(ivb, g0lb, g0rb, g1lb, g1rb, sm):
        pltpu.make_async_copy(acdl_h.at[ivb], g0lb, sm).wait()
        pltpu.make_async_copy(acdr_h.at[ivb], g0rb, sm).wait()
        pltpu.make_async_copy(acdl_h.at[ivb], g1lb, sm).wait()
        pltpu.make_async_copy(acdr_h.at[ivb], g1rb, sm).wait()

    def compute(t, ivb, jvb, g0lb, g0rb, g1lb, g1rb):
        base = (w + t * STRIDE) * CH_C

        @pl.loop(0, CH_C)
        def _row(rr):
            for c0 in range(0, HALF, 16):
                g0lb[rr, pl.ds(c0, 16)] = (
                    g0lb[rr, pl.ds(c0, 16)] * g1lb[rr, pl.ds(c0, 16)])
                g0rb[rr, pl.ds(c0, 16)] = (
                    g0rb[rr, pl.ds(c0, 16)] * g1rb[rr, pl.ds(c0, 16)])

        pltpu.sync_copy(g0lb, bl_h.at[pl.ds(base, CH_C)])
        pltpu.sync_copy(g0rb, br_h.at[pl.ds(base, CH_C)])
        for k in range(CH_C // 16):
            ii = ivb[pl.ds(k * 16, 16)]
            jj = jvb[pl.ds(k * 16, 16)]
            zi = plsc.load_gather(an_v, [ii])
            zj = plsc.load_gather(an_v, [jj])
            pidx = zi * NSPEC + zj
            scb[pl.ds(k * 16, 16)] = plsc.load_gather(sp_v, [pidx])
            shb[pl.ds(k * 16, 16)] = plsc.load_gather(shp_v, [pidx])
        pltpu.sync_copy(scb, scl_h.at[pl.ds(base, CH_C)])
        pltpu.sync_copy(shb, shf_h.at[pl.ds(base, CH_C)])

    def valid(t):
        return w + t * STRIDE < nchunk

    @pl.when(valid(0))
    def _():
        start_idx(0, ivA, jvA, semIA)
        drain_idx(ivA, jvA, semIA)
        fire_g(ivA, jvA, gA0l, gA0r, gA1l, gA1r, semGA)

    @pl.when(valid(1))
    def _():
        start_idx(1, ivB, jvB, semIB)

    @pl.loop(0, itc, step=2)
    def _chunk(t):
        @pl.when(valid(t + 1))
        def _():
            drain_idx(ivB, jvB, semIB)
            fire_g(ivB, jvB, gB0l, gB0r, gB1l, gB1r, semGB)

        @pl.when(valid(t))
        def _():
            drain_g(ivA, gA0l, gA0r, gA1l, gA1r, semGA)
            compute(t, ivA, jvA, gA0l, gA0r, gA1l, gA1r)

        @pl.when(valid(t + 2))
        def _():
            start_idx(t + 2, ivA, jvA, semIA)
            drain_idx(ivA, jvA, semIA)
            fire_g(ivA, jvA, gA0l, gA0r, gA1l, gA1r, semGA)

        @pl.when(valid(t + 1))
        def _():
            drain_g(ivB, gB0l, gB0r, gB1l, gB1r, semGB)
            compute(t + 1, ivB, jvB, gB0l, gB0r, gB1l, gB1r)

        @pl.when(valid(t + 3))
        def _():
            start_idx(t + 3, ivB, jvB, semIB)


def _bc(an_pad, bci, bcj, acdl, acdr, sp_flat, shp_flat):
    ec = bci.shape[0]
    nchunk = ec // CH_C
    itc = -(-nchunk // (NC * NS))
    mesh = plsc.VectorSubcoreMesh(
        core_axis_name="c", subcore_axis_name="s", num_cores=NC, num_subcores=NS)
    f = pl.kernel(
        functools.partial(_bc_body, nchunk, itc),
        out_type=(
            jax.ShapeDtypeStruct((ec, HALF), F32),
            jax.ShapeDtypeStruct((ec, HALF), F32),
            jax.ShapeDtypeStruct((ec,), F32),
            jax.ShapeDtypeStruct((ec,), F32),
        ),
        mesh=mesh,
        compiler_params=pltpu.CompilerParams(needs_layout_passes=False),
        scratch_types=[
            pltpu.VMEM((N_PAD,), I32),
            pltpu.VMEM((NSPEC * NSPEC,), F32),
            pltpu.VMEM((NSPEC * NSPEC,), F32),
            pltpu.VMEM((CH_C,), I32),
            pltpu.VMEM((CH_C,), I32),
            pltpu.VMEM((CH_C,), I32),
            pltpu.VMEM((CH_C,), I32),
            pltpu.VMEM((CH_C, HALF), F32),
            pltpu.VMEM((CH_C, HALF), F32),
            pltpu.VMEM((CH_C, HALF), F32),
            pltpu.VMEM((CH_C, HALF), F32),
            pltpu.VMEM((CH_C, HALF), F32),
            pltpu.VMEM((CH_C, HALF), F32),
            pltpu.VMEM((CH_C, HALF), F32),
            pltpu.VMEM((CH_C, HALF), F32),
            pltpu.VMEM((CH_C,), F32),
            pltpu.VMEM((CH_C,), F32),
            pltpu.SemaphoreType.DMA,
            pltpu.SemaphoreType.DMA,
            pltpu.SemaphoreType.DMA,
            pltpu.SemaphoreType.DMA,
        ],
    )
    return f(an_pad, bci, bcj, acdl, acdr, sp_flat, shp_flat)


def _mlp(x, w1_ref, b1_ref, w2_ref, b2_ref, g_ref, b_ref):
    h = jnp.dot(x, w1_ref[...], preferred_element_type=F32) + b1_ref[...]
    h = h * jax.nn.sigmoid(h)
    o = jnp.dot(h, w2_ref[...], preferred_element_type=F32) + b2_ref[...]
    m = jnp.mean(o, axis=-1, keepdims=True)
    v = jnp.mean((o - m) * (o - m), axis=-1, keepdims=True)
    return (o - m) / jnp.sqrt(v + 1e-5) * g_ref[...] + b_ref[...]


def _off_body(bl_ref, br_ref, disp_ref, scl_ref, shf_ref, wrbf_ref,
              w1_ref, b1_ref, w2_ref, b2_ref, g_ref, b_ref, wo_ref, bo_ref,
              out_ref):
    w, cut = _radial(disp_ref[...], wrbf_ref)
    x = jnp.concatenate([bl_ref[...], br_ref[...]], axis=1) * (w * cut)
    o = _mlp(x, w1_ref, b1_ref, w2_ref, b2_ref, g_ref, b_ref)
    irr = jnp.dot(o, wo_ref[...], preferred_element_type=F32) + bo_ref[...]
    out_ref[...] = irr * scl_ref[...] + shf_ref[...] * cut


def _off(bl, br, disp_pad, scl, shf, wrbf, w1, b1, w2, b2, g, b, wo, bo):
    full = lambda r, c: pl.BlockSpec((r, c), lambda i: (0, 0))
    return pl.pallas_call(
        _off_body,
        grid=(bl.shape[0] // EBF,),
        in_specs=[
            pl.BlockSpec((EBF, HALF), lambda i: (i, 0)),
            pl.BlockSpec((EBF, HALF), lambda i: (i, 0)),
            pl.BlockSpec((EBF, 3), lambda i: (i, 0)),
            pl.BlockSpec((EBF, 1), lambda i: (i, 0)),
            pl.BlockSpec((EBF, 1), lambda i: (i, 0)),
            full(NRBF, D), full(D, H), full(1, H), full(H, D), full(1, D),
            full(1, D), full(1, D), full(D, DOFF), full(1, DOFF),
        ],
        out_specs=pl.BlockSpec((EBF, DOFF), lambda i: (i, 0)),
        out_shape=jax.ShapeDtypeStruct((bl.shape[0], DOFF), F32),
    )(bl, br, disp_pad, scl, shf, wrbf, w1, b1, w2, b2, g, b, wo, bo)


def _on_body(al_ref, ar_ref, son_ref, shon_ref,
             w1_ref, b1_ref, w2_ref, b2_ref, g_ref, b_ref, wo_ref, bo_ref,
             out_ref):
    x = jnp.concatenate([al_ref[...], ar_ref[...]], axis=1)
    o = _mlp(x, w1_ref, b1_ref, w2_ref, b2_ref, g_ref, b_ref)
    irr = jnp.dot(o, wo_ref[...], preferred_element_type=F32) + bo_ref[...]
    out_ref[...] = irr * son_ref[...] + shon_ref[...]


def _on(al, ar, son, shon, w1, b1, w2, b2, g, b, wo, bo):
    full = lambda r, c: pl.BlockSpec((r, c), lambda i: (0, 0))
    return pl.pallas_call(
        _on_body,
        grid=(N_PAD // EB,),
        in_specs=[
            pl.BlockSpec((EB, HALF), lambda i: (i, 0)),
            pl.BlockSpec((EB, HALF), lambda i: (i, 0)),
            pl.BlockSpec((EB, 1), lambda i: (i, 0)),
            pl.BlockSpec((EB, 1), lambda i: (i, 0)),
            full(D, H), full(1, H), full(H, D), full(1, D),
            full(1, D), full(1, D), full(D, DON), full(1, DON),
        ],
        out_specs=pl.BlockSpec((EB, DON), lambda i: (i, 0)),
        out_shape=jax.ShapeDtypeStruct((N_PAD, DON), F32),
    )(al, ar, son, shon, w1, b1, w2, b2, g, b, wo, bo)


def kernel(atomic_numbers, bc_neighbour_indices, bc_neighbour_displacements,
           ac_neighbour_indices, ac_neighbour_displacements, Z_table,
           W_rbf_ac, W_rbf_bc, W1, b1, W2, b2, ln1_g, ln1_b, ln2_g, ln2_b,
           W_off, b_off, W_on, b_on, scale_pair, shift_pair, scale_on,
           shift_on):
    an = atomic_numbers.astype(I32)
    an_pad = jnp.pad(an, (0, N_PAD - N))
    aci = ac_neighbour_indices[:, 0].astype(I32)
    acj = ac_neighbour_indices[:, 1].astype(I32)
    bci = bc_neighbour_indices[:, 0].astype(I32)
    bcj = bc_neighbour_indices[:, 1].astype(I32)
    zl = Z_table[:, :HALF]
    zr = Z_table[:, HALF:]
    zpad = jnp.pad(Z_table, ((0, 128 - NSPEC), (0, 0)))
    sont = jnp.pad(scale_on, (0, 128 - NSPEC))
    shont = jnp.pad(shift_on, (0, 128 - NSPEC))

    zj = _zjk(an_pad, acj)
    ml, mr = _msg(ac_neighbour_displacements, zj.reshape(E, 1), W_rbf_ac, zpad)
    acdl, acdr, son, shon = _acd(an_pad, aci, zl, zr, ml, mr, sont, shont)

    on = _on(acdl, acdr, son.reshape(N_PAD, 1), shon.reshape(N_PAD, 1),
             W1, b1.reshape(1, H), W2, b2.reshape(1, D),
             ln2_g.reshape(1, D), ln2_b.reshape(1, D), W_on,
             b_on.reshape(1, DON))

    sp_flat = scale_pair.reshape(-1)
    shp_flat = shift_pair.reshape(-1)
    ec = E // NSPLIT
    offs = []
    for k in range(NSPLIT):
        sl = slice(k * ec, (k + 1) * ec)
        bl, br, scl, shf = _bc(an_pad, bci[sl], bcj[sl], acdl, acdr,
                               sp_flat, shp_flat)
        offs.append(_off(bl, br, bc_neighbour_displacements[sl],
                         scl.reshape(ec, 1), shf.reshape(ec, 1),
                         W_rbf_bc, W1, b1.reshape(1, H), W2, b2.reshape(1, D),
                         ln1_g.reshape(1, D), ln1_b.reshape(1, D), W_off,
                         b_off.reshape(1, DOFF)))
    off = jnp.concatenate(offs, axis=0) if NSPLIT > 1 else offs[0]
    return off, on[:N]


# speedup vs baseline: 1.0664x; 1.0176x over previous
import functools

import jax
import jax.numpy as jnp
from jax import lax
from jax.experimental import pallas as pl
from jax.experimental.pallas import tpu as pltpu
from jax.experimental.pallas import tpu_sc as plsc

N = 10000
E = 160000
D = 256
HALF = 128
H = 512
NRBF = 16
NSPEC = 100
DOFF = 64
DON = 64
CUTOFF = 5.0
F32 = jnp.float32
I32 = jnp.int32

NC, NS = 2, 16
N_PAD = 10240
NODES_PER_TILE = N_PAD // NS
E_PER_TILE_A = E // NS
CH_A = 80
IT_A = E_PER_TILE_A // CH_A
CH_C = 64
NCHUNK_C = E // CH_C
IT_C = -(-NCHUNK_C // (NC * NS))
EB = 512
EBF = 1600
NSPLIT = 1


def _axis(name):
    return lax.axis_index(name)


def _gather_rows(tbl_h, idx_ref, dst, sem):
    pltpu.async_copy(tbl_h.at[idx_ref], dst, sem).wait()


def _gather_start(tbl_h, idx_ref, dst, sem):
    return pltpu.async_copy(tbl_h.at[idx_ref], dst, sem)


def _scatter_add_rows(src, acc, idx_ref):
    pltpu.sync_copy(src, acc.at[idx_ref], add=True)


def _cutoff_fn(r):
    return jnp.where(r < CUTOFF, 0.5 * (jnp.cos(jnp.pi * r / CUTOFF) + 1.0), 0.0)


def _radial(disp, wrbf_ref):
    r = jnp.sqrt(jnp.sum(disp * disp, axis=1, keepdims=True))
    mu = lax.broadcasted_iota(I32, (1, NRBF), 1).astype(F32) * (CUTOFF / (NRBF - 1))
    phi = jnp.exp(-10.0 * (r - mu) ** 2)
    cut = _cutoff_fn(r)
    w = jnp.dot(phi, wrbf_ref[...], preferred_element_type=F32)
    return w, cut


EPW = E // (NC * NS)
EPW_PAD = EPW + 8


def _zjk_body(an_h, acj_h, zj_h, an_v, jv, zb):
    c = _axis("c")
    s = _axis("s")
    w = s * NC + c
    pltpu.sync_copy(an_h, an_v)
    base = w * EPW
    pltpu.sync_copy(acj_h.at[pl.ds(base, EPW)], jv.at[pl.ds(0, EPW)])
    lane = lax.iota(I32, 16)

    @pl.loop(0, EPW_PAD // 16)
    def _(k):
        off = k * 16
        jj = jv[pl.ds(off, 16)]
        jj = jnp.where(lane < EPW - off, jj, 0)
        zb[pl.ds(off, 16)] = plsc.load_gather(an_v, [jj])

    pltpu.sync_copy(zb.at[pl.ds(0, EPW)], zj_h.at[pl.ds(base, EPW)])


def _zjk(an_pad, acj):
    mesh = plsc.VectorSubcoreMesh(
        core_axis_name="c", subcore_axis_name="s", num_cores=NC, num_subcores=NS)
    f = pl.kernel(
        _zjk_body,
        out_type=jax.ShapeDtypeStruct((E,), I32),
        mesh=mesh,
        compiler_params=pltpu.CompilerParams(needs_layout_passes=False),
        scratch_types=[
            pltpu.VMEM((N_PAD,), I32),
            pltpu.VMEM((EPW_PAD,), I32),
            pltpu.VMEM((EPW_PAD,), I32),
        ],
    )
    return f(an_pad, acj)


WB = 1600


def _msg_body(disp_ref, zj_ref, wrbf_ref, zpad_ref, out_l_ref, out_r_ref):
    w, cut = _radial(disp_ref[...], wrbf_ref)
    z = zj_ref[...]
    oh = (z == lax.broadcasted_iota(I32, (1, 128), 1)).astype(F32)
    zq = zpad_ref[...]
    hi = lax.bitcast_convert_type(
        lax.bitcast_convert_type(zq, I32) & jnp.int32(-65536), F32)
    lo = zq - hi
    wc = w * cut
    m = (wc * jnp.dot(oh, hi, preferred_element_type=F32)
         + wc * jnp.dot(oh, lo, preferred_element_type=F32))
    out_l_ref[...] = m[:, :HALF]
    out_r_ref[...] = m[:, HALF:]


def _msg(disp, zj2, wrbf, zpad):
    return pl.pallas_call(
        _msg_body,
        grid=(E // WB,),
        in_specs=[
            pl.BlockSpec((WB, 3), lambda i: (i, 0)),
            pl.BlockSpec((WB, 1), lambda i: (i, 0)),
            pl.BlockSpec((NRBF, D), lambda i: (0, 0)),
            pl.BlockSpec((128, D), lambda i: (0, 0)),
        ],
        out_specs=[pl.BlockSpec((WB, HALF), lambda i: (i, 0))] * 2,
        out_shape=[jax.ShapeDtypeStruct((E, HALF), F32)] * 2,
    )(disp, zj2, wrbf, zpad)


def _acd_body(an_h, aci_h, zl_h, zr_h, ml_h, mr_h, sont_h, shont_h,
              acdl_h, acdr_h, son_h, shon_h,
              an_v, sont_v, shont_v, iv0, iv1, iv2, m0, m1, m2,
              sbuf, shbuf, acc, sem0, sem1, sem2):
    c = _axis("c")
    s = _axis("s")
    node_base = s * NODES_PER_TILE

    def half(z_h, m_h, acd_h):
        for k in range(NODES_PER_TILE // CH_A):
            off = node_base + k * CH_A
            pltpu.sync_copy(an_h.at[pl.ds(off, CH_A)], iv0)
            _gather_rows(z_h, iv0, m0, sem0)
            pltpu.sync_copy(m0, acc.at[pl.ds(off, CH_A)])
        plsc.subcore_barrier()
        ebase = s * E_PER_TILE_A

        def start(t, ivb, mb, sm):
            base = ebase + t * CH_A
            pltpu.async_copy(aci_h.at[pl.ds(base, CH_A)], ivb, sm)
            pltpu.async_copy(m_h.at[pl.ds(base, CH_A)], mb, sm)

        def drain(ivb, mb, sm):
            pltpu.make_async_copy(aci_h.at[pl.ds(0, CH_A)], ivb, sm).wait()
            pltpu.make_async_copy(m_h.at[pl.ds(0, CH_A)], mb, sm).wait()

        start(0, iv0, m0, sem0)
        start(1, iv1, m1, sem1)

        @pl.loop(0, IT_A, step=3)
        def _edge_chunk(t):
            drain(iv0, m0, sem0)
            _scatter_add_rows(m0, acc, iv0)

            @pl.when(t + 2 < IT_A)
            def _():
                start(t + 2, iv2, m2, sem2)

            @pl.when(t + 1 < IT_A)
            def _():
                drain(iv1, m1, sem1)
                _scatter_add_rows(m1, acc, iv1)

                @pl.when(t + 3 < IT_A)
                def _():
                    start(t + 3, iv0, m0, sem0)

            @pl.when(t + 2 < IT_A)
            def _():
                drain(iv2, m2, sem2)
                _scatter_add_rows(m2, acc, iv2)

                @pl.when(t + 4 < IT_A)
                def _():
                    start(t + 4, iv1, m1, sem1)

        plsc.subcore_barrier()
        pltpu.sync_copy(acc.at[pl.ds(node_base, NODES_PER_TILE)],
                        acd_h.at[pl.ds(node_base, NODES_PER_TILE)])

    @pl.when(c == 0)
    def _():
        half(zl_h, ml_h, acdl_h)
        pltpu.sync_copy(an_h, an_v)
        pltpu.sync_copy(sont_h, sont_v)
        pltpu.sync_copy(shont_h, shont_v)

        @pl.loop(0, NODES_PER_TILE // 16)
        def _n16(k):
            zn = an_v[pl.ds(node_base + k * 16, 16)]
            sbuf[pl.ds(k * 16, 16)] = plsc.load_gather(sont_v, [zn])
            shbuf[pl.ds(k * 16, 16)] = plsc.load_gather(shont_v, [zn])

        pltpu.sync_copy(sbuf, son_h.at[pl.ds(node_base, NODES_PER_TILE)])
        pltpu.sync_copy(shbuf, shon_h.at[pl.ds(node_base, NODES_PER_TILE)])

    @pl.when(c == 1)
    def _():
        half(zr_h, mr_h, acdr_h)


def _acd(an_pad, aci, zl, zr, ml, mr, sont, shont):
    mesh = plsc.VectorSubcoreMesh(
        core_axis_name="c", subcore_axis_name="s", num_cores=NC, num_subcores=NS)
    f = pl.kernel(
        _acd_body,
        out_type=(
            jax.ShapeDtypeStruct((N_PAD, HALF), F32),
            jax.ShapeDtypeStruct((N_PAD, HALF), F32),
            jax.ShapeDtypeStruct((N_PAD,), F32),
            jax.ShapeDtypeStruct((N_PAD,), F32),
        ),
        mesh=mesh,
        compiler_params=pltpu.CompilerParams(needs_layout_passes=False),
        scratch_types=[
            pltpu.VMEM((N_PAD,), I32),
            pltpu.VMEM((128,), F32),
            pltpu.VMEM((128,), F32),
            pltpu.VMEM((CH_A,), I32),
            pltpu.VMEM((CH_A,), I32),
            pltpu.VMEM((CH_A,), I32),
            pltpu.VMEM((CH_A, HALF), F32),
            pltpu.VMEM((CH_A, HALF), F32),
            pltpu.VMEM((CH_A, HALF), F32),
            pltpu.VMEM((NODES_PER_TILE,), F32),
            pltpu.VMEM((NODES_PER_TILE,), F32),
            pltpu.VMEM_SHARED((N_PAD, HALF), F32),
            pltpu.SemaphoreType.DMA,
            pltpu.SemaphoreType.DMA,
            pltpu.SemaphoreType.DMA,
        ],
    )
    return f(an_pad, aci, zl, zr, ml, mr, sont, shont)


def _bc_body(nchunk, itc, an_h, bci_h, bcj_h, acdl_h, acdr_h, sp_h, shp_h,
             bl_h, br_h, scl_h, shf_h,
             an_v, sp_v, shp_v, ivA, jvA, ivB, jvB,
             gA0l, gA0r, gA1l, gA1r, gB0l, gB0r, gB1l, gB1r,
             scb, shb, semIA, semIB, semGA, semGB):
    c = _axis("c")
    s = _axis("s")
    w = s * NC + c
    pltpu.sync_copy(an_h, an_v)
    pltpu.sync_copy(sp_h, sp_v)
    pltpu.sync_copy(shp_h, shp_v)
    STRIDE = NC * NS

    def start_idx(t, ivb, jvb, sm):
        base = (w + t * STRIDE) * CH_C
        pltpu.async_copy(bci_h.at[pl.ds(base, CH_C)], ivb, sm)
        pltpu.async_copy(bcj_h.at[pl.ds(base, CH_C)], jvb, sm)

    def drain_idx(ivb, jvb, sm):
        pltpu.make_async_copy(bci_h.at[pl.ds(0, CH_C)], ivb, sm).wait()
        pltpu.make_async_copy(bcj_h.at[pl.ds(0, CH_C)], jvb, sm).wait()

    def fire_g(ivb, jvb, g0lb, g0rb, g1lb, g1rb, sm):
        _gather_start(acdl_h, ivb, g0lb, sm)
        _gather_start(acdr_h, ivb, g0rb, sm)
        _gather_start(acdl_h, jvb, g1lb, sm)
        _gather_start(acdr_h, jvb, g1rb, sm)

    def drain_g(ivb, g0lb, g0rb, g1lb, g1rb, sm):
        pltpu.make_async_copy(acdl_h.at[ivb], g0lb, sm).wait()
        pltpu.make_async_copy(acdr_h.at[ivb], g0rb, sm).wait()
        pltpu.make_async_copy(acdl_h.at[ivb], g1lb, sm).wait()
        pltpu.make_async_copy(acdr_h.at[ivb], g1rb, sm).wait()

    def compute(t, ivb, jvb, g0lb, g0rb, g1lb, g1rb):
        base = (w + t * STRIDE) * CH_C

        @pl.loop(0, CH_C)
        def _row(rr):
            for c0 in range(0, HALF, 16):
                g0lb[rr, pl.ds(c0, 16)] = (
                    g0lb[rr, pl.ds(c0, 16)] * g1lb[rr, pl.ds(c0, 16)])
                g0rb[rr, pl.ds(c0, 16)] = (
                    g0rb[rr, pl.ds(c0, 16)] * g1rb[rr, pl.ds(c0, 16)])

        pltpu.sync_copy(g0lb, bl_h.at[pl.ds(base, CH_C)])
        pltpu.sync_copy(g0rb, br_h.at[pl.ds(base, CH_C)])
        for k in range(CH_C // 16):
            ii = ivb[pl.ds(k * 16, 16)]
            jj = jvb[pl.ds(k * 16, 16)]
            zi = plsc.load_gather(an_v, [ii])
            zj = plsc.load_gather(an_v, [jj])
            pidx = zi * NSPEC + zj
            scb[pl.ds(k * 16, 16)] = plsc.load_gather(sp_v, [pidx])
            shb[pl.ds(k * 16, 16)] = plsc.load_gather(shp_v, [pidx])
        pltpu.sync_copy(scb, scl_h.at[pl.ds(base, CH_C)])
        pltpu.sync_copy(shb, shf_h.at[pl.ds(base, CH_C)])

    def valid(t):
        return w + t * STRIDE < nchunk

    @pl.when(valid(0))
    def _():
        start_idx(0, ivA, jvA, semIA)
        drain_idx(ivA, jvA, semIA)
        fire_g(ivA, jvA, gA0l, gA0r, gA1l, gA1r, semGA)

    @pl.when(valid(1))
    def _():
        start_idx(1, ivB, jvB, semIB)

    @pl.loop(0, itc, step=2)
    def _chunk(t):
        @pl.when(valid(t + 1))
        def _():
            drain_idx(ivB, jvB, semIB)
            fire_g(ivB, jvB, gB0l, gB0r, gB1l, gB1r, semGB)

        @pl.when(valid(t))
        def _():
            drain_g(ivA, gA0l, gA0r, gA1l, gA1r, semGA)
            compute(t, ivA, jvA, gA0l, gA0r, gA1l, gA1r)

        @pl.when(valid(t + 2))
        def _():
            start_idx(t + 2, ivA, jvA, semIA)
            drain_idx(ivA, jvA, semIA)
            fire_g(ivA, jvA, gA0l, gA0r, gA1l, gA1r, semGA)

        @pl.when(valid(t + 1))
        def _():
            drain_g(ivB, gB0l, gB0r, gB1l, gB1r, semGB)
            compute(t + 1, ivB, jvB, gB0l, gB0r, gB1l, gB1r)

        @pl.when(valid(t + 3))
        def _():
            start_idx(t + 3, ivB, jvB, semIB)


def _bc(an_pad, bci, bcj, acdl, acdr, sp_flat, shp_flat):
    ec = bci.shape[0]
    nchunk = ec // CH_C
    itc = -(-nchunk // (NC * NS))
    mesh = plsc.VectorSubcoreMesh(
        core_axis_name="c", subcore_axis_name="s", num_cores=NC, num_subcores=NS)
    f = pl.kernel(
        functools.partial(_bc_body, nchunk, itc),
        out_type=(
            jax.ShapeDtypeStruct((ec, HALF), F32),
            jax.ShapeDtypeStruct((ec, HALF), F32),
            jax.ShapeDtypeStruct((ec,), F32),
            jax.ShapeDtypeStruct((ec,), F32),
        ),
        mesh=mesh,
        compiler_params=pltpu.CompilerParams(needs_layout_passes=False),
        scratch_types=[
            pltpu.VMEM((N_PAD,), I32),
            pltpu.VMEM((NSPEC * NSPEC,), F32),
            pltpu.VMEM((NSPEC * NSPEC,), F32),
            pltpu.VMEM((CH_C,), I32),
            pltpu.VMEM((CH_C,), I32),
            pltpu.VMEM((CH_C,), I32),
            pltpu.VMEM((CH_C,), I32),
            pltpu.VMEM((CH_C, HALF), F32),
            pltpu.VMEM((CH_C, HALF), F32),
            pltpu.VMEM((CH_C, HALF), F32),
            pltpu.VMEM((CH_C, HALF), F32),
            pltpu.VMEM((CH_C, HALF), F32),
            pltpu.VMEM((CH_C, HALF), F32),
            pltpu.VMEM((CH_C, HALF), F32),
            pltpu.VMEM((CH_C, HALF), F32),
            pltpu.VMEM((CH_C,), F32),
            pltpu.VMEM((CH_C,), F32),
            pltpu.SemaphoreType.DMA,
            pltpu.SemaphoreType.DMA,
            pltpu.SemaphoreType.DMA,
            pltpu.SemaphoreType.DMA,
        ],
    )
    return f(an_pad, bci, bcj, acdl, acdr, sp_flat, shp_flat)


def _mlp(x, w1_ref, b1_ref, w2_ref, b2_ref, g_ref, b_ref):
    h = jnp.dot(x, w1_ref[...], preferred_element_type=F32) + b1_ref[...]
    h = h * jax.nn.sigmoid(h)
    o = jnp.dot(h, w2_ref[...], preferred_element_type=F32) + b2_ref[...]
    m = jnp.mean(o, axis=-1, keepdims=True)
    v = jnp.mean((o - m) * (o - m), axis=-1, keepdims=True)
    return (o - m) / jnp.sqrt(v + 1e-5) * g_ref[...] + b_ref[...]


def _off_body(bl_ref, br_ref, disp_ref, scl_ref, shf_ref, wrbf_ref,
              w1_ref, b1_ref, w2_ref, b2_ref, g_ref, b_ref, wo_ref, bo_ref,
              out_ref):
    w, cut = _radial(disp_ref[...], wrbf_ref)
    x = jnp.concatenate([bl_ref[...], br_ref[...]], axis=1) * (w * cut)
    o = _mlp(x, w1_ref, b1_ref, w2_ref, b2_ref, g_ref, b_ref)
    irr = jnp.dot(o, wo_ref[...], preferred_element_type=F32) + bo_ref[...]
    out_ref[...] = irr * scl_ref[...] + shf_ref[...] * cut


def _off(bl, br, disp_pad, scl, shf, wrbf, w1, b1, w2, b2, g, b, wo, bo):
    full = lambda r, c: pl.BlockSpec((r, c), lambda i: (0, 0))
    return pl.pallas_call(
        _off_body,
        grid=(bl.shape[0] // EBF,),
        in_specs=[
            pl.BlockSpec((EBF, HALF), lambda i: (i, 0)),
            pl.BlockSpec((EBF, HALF), lambda i: (i, 0)),
            pl.BlockSpec((EBF, 3), lambda i: (i, 0)),
            pl.BlockSpec((EBF, 1), lambda i: (i, 0)),
            pl.BlockSpec((EBF, 1), lambda i: (i, 0)),
            full(NRBF, D), full(D, H), full(1, H), full(H, D), full(1, D),
            full(1, D), full(1, D), full(D, DOFF), full(1, DOFF),
        ],
        out_specs=pl.BlockSpec((EBF, DOFF), lambda i: (i, 0)),
        out_shape=jax.ShapeDtypeStruct((bl.shape[0], DOFF), F32),
    )(bl, br, disp_pad, scl, shf, wrbf, w1, b1, w2, b2, g, b, wo, bo)


def _on_body(al_ref, ar_ref, son_ref, shon_ref,
             w1_ref, b1_ref, w2_ref, b2_ref, g_ref, b_ref, wo_ref, bo_ref,
             out_ref):
    x = jnp.concatenate([al_ref[...], ar_ref[...]], axis=1)
    o = _mlp(x, w1_ref, b1_ref, w2_ref, b2_ref, g_ref, b_ref)
    irr = jnp.dot(o, wo_ref[...], preferred_element_type=F32) + bo_ref[...]
    out_ref[...] = irr * son_ref[...] + shon_ref[...]


def _on(al, ar, son, shon, w1, b1, w2, b2, g, b, wo, bo):
    full = lambda r, c: pl.BlockSpec((r, c), lambda i: (0, 0))
    return pl.pallas_call(
        _on_body,
        grid=(N_PAD // EB,),
        in_specs=[
            pl.BlockSpec((EB, HALF), lambda i: (i, 0)),
            pl.BlockSpec((EB, HALF), lambda i: (i, 0)),
            pl.BlockSpec((EB, 1), lambda i: (i, 0)),
            pl.BlockSpec((EB, 1), lambda i: (i, 0)),
            full(D, H), full(1, H), full(H, D), full(1, D),
            full(1, D), full(1, D), full(D, DON), full(1, DON),
        ],
        out_specs=pl.BlockSpec((EB, DON), lambda i: (i, 0)),
        out_shape=jax.ShapeDtypeStruct((N_PAD, DON), F32),
    )(al, ar, son, shon, w1, b1, w2, b2, g, b, wo, bo)


def kernel(atomic_numbers, bc_neighbour_indices, bc_neighbour_displacements,
           ac_neighbour_indices, ac_neighbour_displacements, Z_table,
           W_rbf_ac, W_rbf_bc, W1, b1, W2, b2, ln1_g, ln1_b, ln2_g, ln2_b,
           W_off, b_off, W_on, b_on, scale_pair, shift_pair, scale_on,
           shift_on):
    an = atomic_numbers.astype(I32)
    an_pad = jnp.pad(an, (0, N_PAD - N))
    aci = ac_neighbour_indices[:, 0].astype(I32)
    acj = ac_neighbour_indices[:, 1].astype(I32)
    bci = bc_neighbour_indices[:, 0].astype(I32)
    bcj = bc_neighbour_indices[:, 1].astype(I32)
    zl = Z_table[:, :HALF]
    zr = Z_table[:, HALF:]
    zpad = jnp.pad(Z_table, ((0, 128 - NSPEC), (0, 0)))
    sont = jnp.pad(scale_on, (0, 128 - NSPEC))
    shont = jnp.pad(shift_on, (0, 128 - NSPEC))

    zj = _zjk(an_pad, acj)
    ml, mr = _msg(ac_neighbour_displacements, zj.reshape(E, 1), W_rbf_ac, zpad)
    acdl, acdr, son, shon = _acd(an_pad, aci, zl, zr, ml, mr, sont, shont)

    on = _on(acdl, acdr, son.reshape(N_PAD, 1), shon.reshape(N_PAD, 1),
             W1, b1.reshape(1, H), W2, b2.reshape(1, D),
             ln2_g.reshape(1, D), ln2_b.reshape(1, D), W_on,
             b_on.reshape(1, DON))

    sp_flat = scale_pair.reshape(-1)
    shp_flat = shift_pair.reshape(-1)
    ec = E // NSPLIT
    offs = []
    for k in range(NSPLIT):
        sl = slice(k * ec, (k + 1) * ec)
        bl, br, scl, shf = _bc(an_pad, bci[sl], bcj[sl], acdl, acdr,
                               sp_flat, shp_flat)
        offs.append(_off(bl, br, bc_neighbour_displacements[sl],
                         scl.reshape(ec, 1), shf.reshape(ec, 1),
                         W_rbf_bc, W1, b1.reshape(1, H), W2, b2.reshape(1, D),
                         ln1_g.reshape(1, D), ln1_b.reshape(1, D), W_off,
                         b_off.reshape(1, DOFF)))
    off = jnp.concatenate(offs, axis=0) if NSPLIT > 1 else offs[0]
    return off, on[:N]


# speedup vs baseline: 1.0770x; 1.0099x over previous
import functools

import jax
import jax.numpy as jnp
from jax import lax
from jax.experimental import pallas as pl
from jax.experimental.pallas import tpu as pltpu
from jax.experimental.pallas import tpu_sc as plsc

N = 10000
E = 160000
D = 256
HALF = 128
H = 512
NRBF = 16
NSPEC = 100
DOFF = 64
DON = 64
CUTOFF = 5.0
F32 = jnp.float32
I32 = jnp.int32

NC, NS = 2, 16
N_PAD = 10240
NODES_PER_TILE = N_PAD // NS
E_PER_TILE_A = E // NS
CH_A = 80
IT_A = E_PER_TILE_A // CH_A
CH_C = 64
NCHUNK_C = E // CH_C
IT_C = -(-NCHUNK_C // (NC * NS))
EB = 512
EBF = 3200
NSPLIT = 1


def _axis(name):
    return lax.axis_index(name)


def _gather_rows(tbl_h, idx_ref, dst, sem):
    pltpu.async_copy(tbl_h.at[idx_ref], dst, sem).wait()


def _gather_start(tbl_h, idx_ref, dst, sem):
    return pltpu.async_copy(tbl_h.at[idx_ref], dst, sem)


def _scatter_add_rows(src, acc, idx_ref):
    pltpu.sync_copy(src, acc.at[idx_ref], add=True)


def _cutoff_fn(r):
    return jnp.where(r < CUTOFF, 0.5 * (jnp.cos(jnp.pi * r / CUTOFF) + 1.0), 0.0)


def _radial(disp, wrbf_ref):
    r = jnp.sqrt(jnp.sum(disp * disp, axis=1, keepdims=True))
    mu = lax.broadcasted_iota(I32, (1, NRBF), 1).astype(F32) * (CUTOFF / (NRBF - 1))
    phi = jnp.exp(-10.0 * (r - mu) ** 2)
    cut = _cutoff_fn(r)
    w = jnp.dot(phi, wrbf_ref[...], preferred_element_type=F32)
    return w, cut


EPW = E // (NC * NS)
EPW_PAD = EPW + 8


def _zjk_body(an_h, acj_h, zj_h, an_v, jv, zb):
    c = _axis("c")
    s = _axis("s")
    w = s * NC + c
    pltpu.sync_copy(an_h, an_v)
    base = w * EPW
    pltpu.sync_copy(acj_h.at[pl.ds(base, EPW)], jv.at[pl.ds(0, EPW)])
    lane = lax.iota(I32, 16)

    @pl.loop(0, EPW_PAD // 16)
    def _(k):
        off = k * 16
        jj = jv[pl.ds(off, 16)]
        jj = jnp.where(lane < EPW - off, jj, 0)
        zb[pl.ds(off, 16)] = plsc.load_gather(an_v, [jj])

    pltpu.sync_copy(zb.at[pl.ds(0, EPW)], zj_h.at[pl.ds(base, EPW)])


def _zjk(an_pad, acj):
    mesh = plsc.VectorSubcoreMesh(
        core_axis_name="c", subcore_axis_name="s", num_cores=NC, num_subcores=NS)
    f = pl.kernel(
        _zjk_body,
        out_type=jax.ShapeDtypeStruct((E,), I32),
        mesh=mesh,
        compiler_params=pltpu.CompilerParams(needs_layout_passes=False),
        scratch_types=[
            pltpu.VMEM((N_PAD,), I32),
            pltpu.VMEM((EPW_PAD,), I32),
            pltpu.VMEM((EPW_PAD,), I32),
        ],
    )
    return f(an_pad, acj)


WB = 3200


def _msg_body(disp_ref, zj_ref, wrbf_ref, zpad_ref, out_l_ref, out_r_ref):
    w, cut = _radial(disp_ref[...], wrbf_ref)
    z = zj_ref[...]
    oh = (z == lax.broadcasted_iota(I32, (1, 128), 1)).astype(F32)
    zq = zpad_ref[...]
    hi = lax.bitcast_convert_type(
        lax.bitcast_convert_type(zq, I32) & jnp.int32(-65536), F32)
    lo = zq - hi
    wc = w * cut
    m = (wc * jnp.dot(oh, hi, preferred_element_type=F32)
         + wc * jnp.dot(oh, lo, preferred_element_type=F32))
    out_l_ref[...] = m[:, :HALF]
    out_r_ref[...] = m[:, HALF:]


def _msg(disp, zj2, wrbf, zpad):
    return pl.pallas_call(
        _msg_body,
        grid=(E // WB,),
        in_specs=[
            pl.BlockSpec((WB, 3), lambda i: (i, 0)),
            pl.BlockSpec((WB, 1), lambda i: (i, 0)),
            pl.BlockSpec((NRBF, D), lambda i: (0, 0)),
            pl.BlockSpec((128, D), lambda i: (0, 0)),
        ],
        out_specs=[pl.BlockSpec((WB, HALF), lambda i: (i, 0))] * 2,
        out_shape=[jax.ShapeDtypeStruct((E, HALF), F32)] * 2,
    )(disp, zj2, wrbf, zpad)


def _acd_body(an_h, aci_h, zl_h, zr_h, ml_h, mr_h, sont_h, shont_h,
              acdl_h, acdr_h, son_h, shon_h,
              an_v, sont_v, shont_v, iv0, iv1, iv2, m0, m1, m2,
              sbuf, shbuf, acc, sem0, sem1, sem2):
    c = _axis("c")
    s = _axis("s")
    node_base = s * NODES_PER_TILE

    def half(z_h, m_h, acd_h):
        for k in range(NODES_PER_TILE // CH_A):
            off = node_base + k * CH_A
            pltpu.sync_copy(an_h.at[pl.ds(off, CH_A)], iv0)
            _gather_rows(z_h, iv0, m0, sem0)
            pltpu.sync_copy(m0, acc.at[pl.ds(off, CH_A)])
        plsc.subcore_barrier()
        ebase = s * E_PER_TILE_A

        def start(t, ivb, mb, sm):
            base = ebase + t * CH_A
            pltpu.async_copy(aci_h.at[pl.ds(base, CH_A)], ivb, sm)
            pltpu.async_copy(m_h.at[pl.ds(base, CH_A)], mb, sm)

        def drain(ivb, mb, sm):
            pltpu.make_async_copy(aci_h.at[pl.ds(0, CH_A)], ivb, sm).wait()
            pltpu.make_async_copy(m_h.at[pl.ds(0, CH_A)], mb, sm).wait()

        start(0, iv0, m0, sem0)
        start(1, iv1, m1, sem1)

        @pl.loop(0, IT_A, step=3)
        def _edge_chunk(t):
            drain(iv0, m0, sem0)
            _scatter_add_rows(m0, acc, iv0)

            @pl.when(t + 2 < IT_A)
            def _():
                start(t + 2, iv2, m2, sem2)

            @pl.when(t + 1 < IT_A)
            def _():
                drain(iv1, m1, sem1)
                _scatter_add_rows(m1, acc, iv1)

                @pl.when(t + 3 < IT_A)
                def _():
                    start(t + 3, iv0, m0, sem0)

            @pl.when(t + 2 < IT_A)
            def _():
                drain(iv2, m2, sem2)
                _scatter_add_rows(m2, acc, iv2)

                @pl.when(t + 4 < IT_A)
                def _():
                    start(t + 4, iv1, m1, sem1)

        plsc.subcore_barrier()
        pltpu.sync_copy(acc.at[pl.ds(node_base, NODES_PER_TILE)],
                        acd_h.at[pl.ds(node_base, NODES_PER_TILE)])

    @pl.when(c == 0)
    def _():
        half(zl_h, ml_h, acdl_h)
        pltpu.sync_copy(an_h, an_v)
        pltpu.sync_copy(sont_h, sont_v)
        pltpu.sync_copy(shont_h, shont_v)

        @pl.loop(0, NODES_PER_TILE // 16)
        def _n16(k):
            zn = an_v[pl.ds(node_base + k * 16, 16)]
            sbuf[pl.ds(k * 16, 16)] = plsc.load_gather(sont_v, [zn])
            shbuf[pl.ds(k * 16, 16)] = plsc.load_gather(shont_v, [zn])

        pltpu.sync_copy(sbuf, son_h.at[pl.ds(node_base, NODES_PER_TILE)])
        pltpu.sync_copy(shbuf, shon_h.at[pl.ds(node_base, NODES_PER_TILE)])

    @pl.when(c == 1)
    def _():
        half(zr_h, mr_h, acdr_h)


def _acd(an_pad, aci, zl, zr, ml, mr, sont, shont):
    mesh = plsc.VectorSubcoreMesh(
        core_axis_name="c", subcore_axis_name="s", num_cores=NC, num_subcores=NS)
    f = pl.kernel(
        _acd_body,
        out_type=(
            jax.ShapeDtypeStruct((N_PAD, HALF), F32),
            jax.ShapeDtypeStruct((N_PAD, HALF), F32),
            jax.ShapeDtypeStruct((N_PAD,), F32),
            jax.ShapeDtypeStruct((N_PAD,), F32),
        ),
        mesh=mesh,
        compiler_params=pltpu.CompilerParams(needs_layout_passes=False),
        scratch_types=[
            pltpu.VMEM((N_PAD,), I32),
            pltpu.VMEM((128,), F32),
            pltpu.VMEM((128,), F32),
            pltpu.VMEM((CH_A,), I32),
            pltpu.VMEM((CH_A,), I32),
            pltpu.VMEM((CH_A,), I32),
            pltpu.VMEM((CH_A, HALF), F32),
            pltpu.VMEM((CH_A, HALF), F32),
            pltpu.VMEM((CH_A, HALF), F32),
            pltpu.VMEM((NODES_PER_TILE,), F32),
            pltpu.VMEM((NODES_PER_TILE,), F32),
            pltpu.VMEM_SHARED((N_PAD, HALF), F32),
            pltpu.SemaphoreType.DMA,
            pltpu.SemaphoreType.DMA,
            pltpu.SemaphoreType.DMA,
        ],
    )
    return f(an_pad, aci, zl, zr, ml, mr, sont, shont)


def _bc_body(nchunk, itc, an_h, bci_h, bcj_h, acdl_h, acdr_h, sp_h, shp_h,
             bl_h, br_h, scl_h, shf_h,
             an_v, sp_v, shp_v, ivA, jvA, ivB, jvB,
             gA0l, gA0r, gA1l, gA1r, gB0l, gB0r, gB1l, gB1r,
             scb, shb, semIA, semIB, semGA, semGB):
    c = _axis("c")
    s = _axis("s")
    w = s * NC + c
    pltpu.sync_copy(an_h, an_v)
    pltpu.sync_copy(sp_h, sp_v)
    pltpu.sync_copy(shp_h, shp_v)
    STRIDE = NC * NS

    def start_idx(t, ivb, jvb, sm):
        base = (w + t * STRIDE) * CH_C
        pltpu.async_copy(bci_h.at[pl.ds(base, CH_C)], ivb, sm)
        pltpu.async_copy(bcj_h.at[pl.ds(base, CH_C)], jvb, sm)

    def drain_idx(ivb, jvb, sm):
        pltpu.make_async_copy(bci_h.at[pl.ds(0, CH_C)], ivb, sm).wait()
        pltpu.make_async_copy(bcj_h.at[pl.ds(0, CH_C)], jvb, sm).wait()

    def fire_g(ivb, jvb, g0lb, g0rb, g1lb, g1rb, sm):
        _gather_start(acdl_h, ivb, g0lb, sm)
        _gather_start(acdr_h, ivb, g0rb, sm)
        _gather_start(acdl_h, jvb, g1lb, sm)
        _gather_start(acdr_h, jvb, g1rb, sm)

    def drain_g(ivb, g0lb, g0rb, g1lb, g1rb, sm):
        pltpu.make_async_copy(acdl_h.at[ivb], g0lb, sm).wait()
        pltpu.make_async_copy(acdr_h.at[ivb], g0rb, sm).wait()
        pltpu.make_async_copy(acdl_h.at[ivb], g1lb, sm).wait()
        pltpu.make_async_copy(acdr_h.at[ivb], g1rb, sm).wait()

    def compute(t, ivb, jvb, g0lb, g0rb, g1lb, g1rb):
        base = (w + t * STRIDE) * CH_C

        @pl.loop(0, CH_C)
        def _row(rr):
            for c0 in range(0, HALF, 16):
                g0lb[rr, pl.ds(c0, 16)] = (
                    g0lb[rr, pl.ds(c0, 16)] * g1lb[rr, pl.ds(c0, 16)])
                g0rb[rr, pl.ds(c0, 16)] = (
                    g0rb[rr, pl.ds(c0, 16)] * g1rb[rr, pl.ds(c0, 16)])

        pltpu.sync_copy(g0lb, bl_h.at[pl.ds(base, CH_C)])
        pltpu.sync_copy(g0rb, br_h.at[pl.ds(base, CH_C)])
        for k in range(CH_C // 16):
            ii = ivb[pl.ds(k * 16, 16)]
            jj = jvb[pl.ds(k * 16, 16)]
            zi = plsc.load_gather(an_v, [ii])
            zj = plsc.load_gather(an_v, [jj])
            pidx = zi * NSPEC + zj
            scb[pl.ds(k * 16, 16)] = plsc.load_gather(sp_v, [pidx])
            shb[pl.ds(k * 16, 16)] = plsc.load_gather(shp_v, [pidx])
        pltpu.sync_copy(scb, scl_h.at[pl.ds(base, CH_C)])
        pltpu.sync_copy(shb, shf_h.at[pl.ds(base, CH_C)])

    def valid(t):
        return w + t * STRIDE < nchunk

    @pl.when(valid(0))
    def _():
        start_idx(0, ivA, jvA, semIA)
        drain_idx(ivA, jvA, semIA)
        fire_g(ivA, jvA, gA0l, gA0r, gA1l, gA1r, semGA)

    @pl.when(valid(1))
    def _():
        start_idx(1, ivB, jvB, semIB)

    @pl.loop(0, itc, step=2)
    def _chunk(t):
        @pl.when(valid(t + 1))
        def _():
            drain_idx(ivB, jvB, semIB)
            fire_g(ivB, jvB, gB0l, gB0r, gB1l, gB1r, semGB)

        @pl.when(valid(t))
        def _():
            drain_g(ivA, gA0l, gA0r, gA1l, gA1r, semGA)
            compute(t, ivA, jvA, gA0l, gA0r, gA1l, gA1r)

        @pl.when(valid(t + 2))
        def _():
            start_idx(t + 2, ivA, jvA, semIA)
            drain_idx(ivA, jvA, semIA)
            fire_g(ivA, jvA, gA0l, gA0r, gA1l, gA1r, semGA)

        @pl.when(valid(t + 1))
        def _():
            drain_g(ivB, gB0l, gB0r, gB1l, gB1r, semGB)
            compute(t + 1, ivB, jvB, gB0l, gB0r, gB1l, gB1r)

        @pl.when(valid(t + 3))
        def _():
            start_idx(t + 3, ivB, jvB, semIB)


def _bc(an_pad, bci, bcj, acdl, acdr, sp_flat, shp_flat):
    ec = bci.shape[0]
    nchunk = ec // CH_C
    itc = -(-nchunk // (NC * NS))
    mesh = plsc.VectorSubcoreMesh(
        core_axis_name="c", subcore_axis_name="s", num_cores=NC, num_subcores=NS)
    f = pl.kernel(
        functools.partial(_bc_body, nchunk, itc),
        out_type=(
            jax.ShapeDtypeStruct((ec, HALF), F32),
            jax.ShapeDtypeStruct((ec, HALF), F32),
            jax.ShapeDtypeStruct((ec,), F32),
            jax.ShapeDtypeStruct((ec,), F32),
        ),
        mesh=mesh,
        compiler_params=pltpu.CompilerParams(needs_layout_passes=False),
        scratch_types=[
            pltpu.VMEM((N_PAD,), I32),
            pltpu.VMEM((NSPEC * NSPEC,), F32),
            pltpu.VMEM((NSPEC * NSPEC,), F32),
            pltpu.VMEM((CH_C,), I32),
            pltpu.VMEM((CH_C,), I32),
            pltpu.VMEM((CH_C,), I32),
            pltpu.VMEM((CH_C,), I32),
            pltpu.VMEM((CH_C, HALF), F32),
            pltpu.VMEM((CH_C, HALF), F32),
            pltpu.VMEM((CH_C, HALF), F32),
            pltpu.VMEM((CH_C, HALF), F32),
            pltpu.VMEM((CH_C, HALF), F32),
            pltpu.VMEM((CH_C, HALF), F32),
            pltpu.VMEM((CH_C, HALF), F32),
            pltpu.VMEM((CH_C, HALF), F32),
            pltpu.VMEM((CH_C,), F32),
            pltpu.VMEM((CH_C,), F32),
            pltpu.SemaphoreType.DMA,
            pltpu.SemaphoreType.DMA,
            pltpu.SemaphoreType.DMA,
            pltpu.SemaphoreType.DMA,
        ],
    )
    return f(an_pad, bci, bcj, acdl, acdr, sp_flat, shp_flat)


def _mlp(x, w1_ref, b1_ref, w2_ref, b2_ref, g_ref, b_ref):
    h = jnp.dot(x, w1_ref[...], preferred_element_type=F32) + b1_ref[...]
    h = h * jax.nn.sigmoid(h)
    o = jnp.dot(h, w2_ref[...], preferred_element_type=F32) + b2_ref[...]
    m = jnp.mean(o, axis=-1, keepdims=True)
    v = jnp.mean((o - m) * (o - m), axis=-1, keepdims=True)
    return (o - m) / jnp.sqrt(v + 1e-5) * g_ref[...] + b_ref[...]


def _off_body(bl_ref, br_ref, disp_ref, scl_ref, shf_ref, wrbf_ref,
              w1_ref, b1_ref, w2_ref, b2_ref, g_ref, b_ref, wo_ref, bo_ref,
              out_ref):
    w, cut = _radial(disp_ref[...], wrbf_ref)
    x = jnp.concatenate([bl_ref[...], br_ref[...]], axis=1) * (w * cut)
    o = _mlp(x, w1_ref, b1_ref, w2_ref, b2_ref, g_ref, b_ref)
    irr = jnp.dot(o, wo_ref[...], preferred_element_type=F32) + bo_ref[...]
    out_ref[...] = irr * scl_ref[...] + shf_ref[...] * cut


def _off(bl, br, disp_pad, scl, shf, wrbf, w1, b1, w2, b2, g, b, wo, bo):
    full = lambda r, c: pl.BlockSpec((r, c), lambda i: (0, 0))
    return pl.pallas_call(
        _off_body,
        grid=(bl.shape[0] // EBF,),
        in_specs=[
            pl.BlockSpec((EBF, HALF), lambda i: (i, 0)),
            pl.BlockSpec((EBF, HALF), lambda i: (i, 0)),
            pl.BlockSpec((EBF, 3), lambda i: (i, 0)),
            pl.BlockSpec((EBF, 1), lambda i: (i, 0)),
            pl.BlockSpec((EBF, 1), lambda i: (i, 0)),
            full(NRBF, D), full(D, H), full(1, H), full(H, D), full(1, D),
            full(1, D), full(1, D), full(D, DOFF), full(1, DOFF),
        ],
        out_specs=pl.BlockSpec((EBF, DOFF), lambda i: (i, 0)),
        out_shape=jax.ShapeDtypeStruct((bl.shape[0], DOFF), F32),
    )(bl, br, disp_pad, scl, shf, wrbf, w1, b1, w2, b2, g, b, wo, bo)


def _on_body(al_ref, ar_ref, son_ref, shon_ref,
             w1_ref, b1_ref, w2_ref, b2_ref, g_ref, b_ref, wo_ref, bo_ref,
             out_ref):
    x = jnp.concatenate([al_ref[...], ar_ref[...]], axis=1)
    o = _mlp(x, w1_ref, b1_ref, w2_ref, b2_ref, g_ref, b_ref)
    irr = jnp.dot(o, wo_ref[...], preferred_element_type=F32) + bo_ref[...]
    out_ref[...] = irr * son_ref[...] + shon_ref[...]


def _on(al, ar, son, shon, w1, b1, w2, b2, g, b, wo, bo):
    full = lambda r, c: pl.BlockSpec((r, c), lambda i: (0, 0))
    return pl.pallas_call(
        _on_body,
        grid=(N_PAD // EB,),
        in_specs=[
            pl.BlockSpec((EB, HALF), lambda i: (i, 0)),
            pl.BlockSpec((EB, HALF), lambda i: (i, 0)),
            pl.BlockSpec((EB, 1), lambda i: (i, 0)),
            pl.BlockSpec((EB, 1), lambda i: (i, 0)),
            full(D, H), full(1, H), full(H, D), full(1, D),
            full(1, D), full(1, D), full(D, DON), full(1, DON),
        ],
        out_specs=pl.BlockSpec((EB, DON), lambda i: (i, 0)),
        out_shape=jax.ShapeDtypeStruct((N_PAD, DON), F32),
    )(al, ar, son, shon, w1, b1, w2, b2, g, b, wo, bo)


def kernel(atomic_numbers, bc_neighbour_indices, bc_neighbour_displacements,
           ac_neighbour_indices, ac_neighbour_displacements, Z_table,
           W_rbf_ac, W_rbf_bc, W1, b1, W2, b2, ln1_g, ln1_b, ln2_g, ln2_b,
           W_off, b_off, W_on, b_on, scale_pair, shift_pair, scale_on,
           shift_on):
    an = atomic_numbers.astype(I32)
    an_pad = jnp.pad(an, (0, N_PAD - N))
    aci = ac_neighbour_indices[:, 0].astype(I32)
    acj = ac_neighbour_indices[:, 1].astype(I32)
    bci = bc_neighbour_indices[:, 0].astype(I32)
    bcj = bc_neighbour_indices[:, 1].astype(I32)
    zl = Z_table[:, :HALF]
    zr = Z_table[:, HALF:]
    zpad = jnp.pad(Z_table, ((0, 128 - NSPEC), (0, 0)))
    sont = jnp.pad(scale_on, (0, 128 - NSPEC))
    shont = jnp.pad(shift_on, (0, 128 - NSPEC))

    zj = _zjk(an_pad, acj)
    ml, mr = _msg(ac_neighbour_displacements, zj.reshape(E, 1), W_rbf_ac, zpad)
    acdl, acdr, son, shon = _acd(an_pad, aci, zl, zr, ml, mr, sont, shont)

    on = _on(acdl, acdr, son.reshape(N_PAD, 1), shon.reshape(N_PAD, 1),
             W1, b1.reshape(1, H), W2, b2.reshape(1, D),
             ln2_g.reshape(1, D), ln2_b.reshape(1, D), W_on,
             b_on.reshape(1, DON))

    sp_flat = scale_pair.reshape(-1)
    shp_flat = shift_pair.reshape(-1)
    ec = E // NSPLIT
    offs = []
    for k in range(NSPLIT):
        sl = slice(k * ec, (k + 1) * ec)
        bl, br, scl, shf = _bc(an_pad, bci[sl], bcj[sl], acdl, acdr,
                               sp_flat, shp_flat)
        offs.append(_off(bl, br, bc_neighbour_displacements[sl],
                         scl.reshape(ec, 1), shf.reshape(ec, 1),
                         W_rbf_bc, W1, b1.reshape(1, H), W2, b2.reshape(1, D),
                         ln1_g.reshape(1, D), ln1_b.reshape(1, D), W_off,
                         b_off.reshape(1, DOFF)))
    off = jnp.concatenate(offs, axis=0) if NSPLIT > 1 else offs[0]
    return off, on[:N]
